# Initial kernel scaffold; baseline (speedup 1.0000x reference)
import jax
import jax.numpy as jnp
from jax.experimental import pallas as pl


def kernel(x, y, y_xyz, idx, Wq, bq, Wk, bk, Wv, bv, geW1, geb1, geg1, gebe1, geW2, geb2, neg0, nebe0, neW1, neb1, neg1, nebe1, neW2, neb2):
    raise NotImplementedError("write your pallas kernel here")


# speedup vs baseline: 7.3642x; 7.3642x over previous
import functools

import jax
import jax.numpy as jnp
from jax import lax
from jax.experimental import pallas as pl
from jax.experimental.pallas import tpu as pltpu
from jax.experimental.pallas import tpu_sc as plsc

B, C, N, S = 2, 128, 10000, 16
QK = 32
NPAD = 10240
F = B * N
FS = F * S
TILE = 400
GRID = F // TILE
TN = 1024
MF = float(FS)
CHUNK = 128
NCHUNK = FS // CHUNK
NW = 32
ITERS = (NCHUNK + NW - 1) // NW

_dims = (((1,), (1,)), ((), ()))


def _pcall(body, **kw):
    return pl.pallas_call(body, **kw)


def _dot(a, b):
    return lax.dot_general(a, b, _dims, preferred_element_type=jnp.float32)


def _proj_body(x_ref, y_ref, xyz_ref, wq_ref, bq_ref, wk_ref, bk_ref,
               wv_ref, bv_ref, xq_ref, tkx_ref, tv_ref):
    xb = x_ref[0]
    yb = y_ref[0]
    d0 = (((0,), (1,)), ((), ()))
    xq_ref[...] = lax.dot_general(xb, wq_ref[...], d0,
                                  preferred_element_type=jnp.float32) + bq_ref[...]
    k = lax.dot_general(yb, wk_ref[...], d0,
                        preferred_element_type=jnp.float32) + bk_ref[...]
    tkx_ref[...] = jnp.concatenate(
        [k, xyz_ref[...], jnp.zeros((TN, 128 - QK - 16), jnp.float32)], axis=1)
    tv_ref[...] = lax.dot_general(yb, wv_ref[...], d0,
                                  preferred_element_type=jnp.float32) + bv_ref[...]


def _projections(x_p, y_p, xyzp, Wq, bq, Wk, bk, Wv, bv):
    nt = NPAD // TN
    return _pcall(
        _proj_body,
        grid=(B, nt),
        in_specs=[
            pl.BlockSpec((1, C, TN), lambda b, i: (b, 0, i)),
            pl.BlockSpec((1, C, TN), lambda b, i: (b, 0, i)),
            pl.BlockSpec((TN, 16), lambda b, i: (b * (NPAD // TN) + i, 0)),
            pl.BlockSpec((QK, C), lambda b, i: (0, 0)),
            pl.BlockSpec((1, QK), lambda b, i: (0, 0)),
            pl.BlockSpec((QK, C), lambda b, i: (0, 0)),
            pl.BlockSpec((1, QK), lambda b, i: (0, 0)),
            pl.BlockSpec((C, C), lambda b, i: (0, 0)),
            pl.BlockSpec((1, C), lambda b, i: (0, 0)),
        ],
        out_specs=[
            pl.BlockSpec((TN, QK), lambda b, i: (b * (NPAD // TN) + i, 0)),
            pl.BlockSpec((TN, C), lambda b, i: (b * (NPAD // TN) + i, 0)),
            pl.BlockSpec((TN, C), lambda b, i: (b * (NPAD // TN) + i, 0)),
        ],
        out_shape=[
            jax.ShapeDtypeStruct((B * NPAD, QK), jnp.float32),
            jax.ShapeDtypeStruct((B * NPAD, C), jnp.float32),
            jax.ShapeDtypeStruct((B * NPAD, C), jnp.float32),
        ],
    )(x_p, y_p, xyzp, Wq, bq, Wk, bk, Wv, bv)


def _sc_gather(gidx, tkx, tv):
    mesh = plsc.VectorSubcoreMesh(core_axis_name="c", subcore_axis_name="s")

    @functools.partial(
        pl.kernel,
        mesh=mesh,
        out_type=[
            jax.ShapeDtypeStruct((FS, C), jnp.float32),
            jax.ShapeDtypeStruct((FS, C), jnp.float32),
        ],
        scratch_types=[
            pltpu.VMEM((CHUNK,), jnp.int32),
            pltpu.VMEM((CHUNK, C), jnp.float32),
            pltpu.VMEM((CHUNK, C), jnp.float32),
            pltpu.SemaphoreType.DMA,
        ],
    )
    def k(gidx_ref, tkx_ref, tv_ref, gkx_ref, gv_ref, idx_v, kbuf, vbuf, sem):
        wid = lax.axis_index("s") * 2 + lax.axis_index("c")

        def body(j, carry):
            cid = wid + j * NW

            @pl.when(cid < NCHUNK)
            def _():
                base = cid * CHUNK
                pltpu.sync_copy(gidx_ref.at[pl.ds(base, CHUNK)], idx_v)
                g1 = pltpu.async_copy(tkx_ref.at[idx_v], kbuf, sem)
                g2 = pltpu.async_copy(tv_ref.at[idx_v], vbuf, sem)
                g1.wait()
                g2.wait()
                s1 = pltpu.async_copy(kbuf, gkx_ref.at[pl.ds(base, CHUNK)], sem)
                s2 = pltpu.async_copy(vbuf, gv_ref.at[pl.ds(base, CHUNK)], sem)
                s1.wait()
                s2.wait()

            return carry

        lax.fori_loop(0, ITERS, body, 0)

    return k(gidx, tkx, tv)


def _geo_u16(gx, cx):
    diff = gx - cx[:, None, :]
    sq = jnp.sum(diff * diff, axis=-1, keepdims=True)
    euc = jnp.sqrt(sq + 1e-12)
    lane = lax.broadcasted_iota(jnp.int32, diff.shape, 2)
    return jnp.where(lane == 3, euc, diff)


def _affine(s_ref, gamma, beta):
    mu = s_ref[0:1, :] / MF
    var = s_ref[1:2, :] / MF - mu * mu
    a = gamma * lax.rsqrt(var + 1e-5)
    return a, beta - mu * a


def _hpre(kx, cx, gw1, gb1):
    u = _geo_u16(kx[:, :, QK:QK + 16], cx).reshape(TILE * S, 16)
    return _dot(u, gw1) + gb1


def _emb_of(kx, cx, xq, s1_ref, gw1, gb1, gg1, gbe1, gw2, gb2):
    hp = _hpre(kx, cx, gw1, gb1)
    a1, c1 = _affine(s1_ref, gg1, gbe1)
    h1 = jnp.maximum(hp * a1 + c1, 0.0)
    h = _dot(h1, gw2) + gb2
    d = kx[:, :, 0:QK] - xq[:, None, :]
    return h - (d * d).reshape(TILE * S, QK)


def _acc_stats(s_ref, v):
    @pl.when(pl.program_id(0) == 0)
    def _():
        s_ref[...] = jnp.zeros_like(s_ref)

    s_ref[0:1, :] += jnp.sum(v, axis=0)[None]
    s_ref[1:2, :] += jnp.sum(v * v, axis=0)[None]


def _c1_body(kx_ref, tx_ref, gw1_ref, gb1_ref, s1_ref):
    _acc_stats(s1_ref, _hpre(kx_ref[...], tx_ref[...], gw1_ref[...],
                             gb1_ref[...]))


def _c2_body(kx_ref, tx_ref, xq_ref, s1_ref, gw1_ref, gb1_ref,
             gg1_ref, gbe1_ref, gw2_ref, gb2_ref, s2_ref):
    emb = _emb_of(kx_ref[...], tx_ref[...], xq_ref[...], s1_ref,
                  gw1_ref[...], gb1_ref[...], gg1_ref[...], gbe1_ref[...],
                  gw2_ref[...], gb2_ref[...])
    _acc_stats(s2_ref, emb)


def _c3_body(kx_ref, tx_ref, xq_ref, s1_ref, gw1_ref, gb1_ref,
             gg1_ref, gbe1_ref, gw2_ref, gb2_ref, s2_ref, ng0_ref, nbe0_ref,
             nw1_ref, nb1_ref, s3_ref):
    emb = _emb_of(kx_ref[...], tx_ref[...], xq_ref[...], s1_ref,
                  gw1_ref[...], gb1_ref[...], gg1_ref[...], gbe1_ref[...],
                  gw2_ref[...], gb2_ref[...])
    a2, c2 = _affine(s2_ref, ng0_ref[...], nbe0_ref[...])
    r = jnp.maximum(emb * a2 + c2, 0.0)
    t1 = _dot(r, nw1_ref[...]) + nb1_ref[...]
    _acc_stats(s3_ref, t1)


def _c4_body(kx_ref, tx_ref, xq_ref, gv_ref, s1_ref, s2_ref, s3_ref,
             gw1_ref, gb1_ref, gg1_ref, gbe1_ref, gw2_ref, gb2_ref,
             ng0_ref, nbe0_ref, nw1_ref, nb1_ref, ng1_ref, nbe1_ref,
             nw2_ref, nb2_ref, out_ref):
    emb = _emb_of(kx_ref[...], tx_ref[...], xq_ref[...], s1_ref,
                  gw1_ref[...], gb1_ref[...], gg1_ref[...], gbe1_ref[...],
                  gw2_ref[...], gb2_ref[...])
    a2, c2 = _affine(s2_ref, ng0_ref[...], nbe0_ref[...])
    r = jnp.maximum(emb * a2 + c2, 0.0)
    t1 = _dot(r, nw1_ref[...]) + nb1_ref[...]
    a3, c3 = _affine(s3_ref, ng1_ref[...], nbe1_ref[...])
    t1 = jnp.maximum(t1 * a3 + c3, 0.0)
    t2 = (_dot(t1, nw2_ref[...]) + nb2_ref[...]).reshape(TILE, S, C)
    m = jnp.max(t2, axis=1, keepdims=True)
    e = jnp.exp(t2 - m)
    w = e / jnp.sum(e, axis=1, keepdims=True)
    out_ref[...] = jnp.sum(gv_ref[...] * w, axis=1)


def _full(shape):
    return pl.BlockSpec(shape, lambda i: tuple(0 for _ in shape))


def kernel(x, y, y_xyz, idx, Wq, bq, Wk, bk, Wv, bv, geW1, geb1, geg1, gebe1,
           geW2, geb2, neg0, nebe0, neW1, neb1, neg1, nebe1, neW2, neb2):
    f32 = jnp.float32
    x_p = jnp.pad(x, ((0, 0), (0, 0), (0, NPAD - N)))
    y_p = jnp.pad(y, ((0, 0), (0, 0), (0, NPAD - N)))

    xyzt = jnp.transpose(y_xyz, (0, 2, 1))
    txp = jnp.pad(xyzt, ((0, 0), (0, NPAD - N), (0, 13))).reshape(B * NPAD, 16)
    txc = jnp.pad(xyzt, ((0, 0), (0, 0), (0, 13))).reshape(F, 16)

    xqp, tkx, tv = _projections(
        x_p, y_p, txp, Wq, bq.reshape(1, QK).astype(f32), Wk,
        bk.reshape(1, QK).astype(f32), Wv, bv.reshape(1, C).astype(f32))
    xq = xqp.reshape(B, NPAD, QK)[:, :N].reshape(F, QK)

    gidx = (idx.astype(jnp.int32)
            + (jnp.arange(B, dtype=jnp.int32) * NPAD)[:, None, None]
            ).reshape(FS)

    gkx, gv = _sc_gather(gidx, tkx, tv)
    kx3 = gkx.reshape(F, S, C)
    gv3 = gv.reshape(F, S, C)

    gw1p = jnp.pad(geW1, ((0, 0), (0, 12)))
    r32 = lambda v: v.reshape(1, QK)
    stats_spec = pl.BlockSpec((8, QK), lambda i: (0, 0))
    stats_shape = jax.ShapeDtypeStruct((8, QK), f32)
    kx_spec = pl.BlockSpec((TILE, S, C), lambda i: (i, 0, 0))
    tx_spec = pl.BlockSpec((TILE, 16), lambda i: (i, 0))
    xq_spec = pl.BlockSpec((TILE, QK), lambda i: (i, 0))

    s1 = _pcall(
        _c1_body,
        grid=(GRID,),
        in_specs=[kx_spec, tx_spec, _full((QK, 16)), _full((1, QK))],
        out_specs=stats_spec,
        out_shape=stats_shape,
    )(kx3, txc, gw1p, r32(geb1))

    ge_args = (gw1p, r32(geb1), r32(geg1), r32(gebe1), geW2, r32(geb2))
    ge_specs = [_full((QK, 16)), _full((1, QK)), _full((1, QK)),
                _full((1, QK)), _full((QK, QK)), _full((1, QK))]

    s2 = _pcall(
        _c2_body,
        grid=(GRID,),
        in_specs=[kx_spec, tx_spec, xq_spec, stats_spec] + ge_specs,
        out_specs=stats_spec,
        out_shape=stats_shape,
    )(kx3, txc, xq, s1, *ge_args)

    s3 = _pcall(
        _c3_body,
        grid=(GRID,),
        in_specs=([kx_spec, tx_spec, xq_spec, stats_spec] + ge_specs
                  + [stats_spec, _full((1, QK)), _full((1, QK)),
                     _full((QK, QK)), _full((1, QK))]),
        out_specs=stats_spec,
        out_shape=stats_shape,
    )(kx3, txc, xq, s1, *ge_args, s2, r32(neg0), r32(nebe0), neW1, r32(neb1))

    out = _pcall(
        _c4_body,
        grid=(GRID,),
        in_specs=([kx_spec, tx_spec, xq_spec,
                   pl.BlockSpec((TILE, S, C), lambda i: (i, 0, 0)),
                   stats_spec, stats_spec, stats_spec] + ge_specs
                  + [_full((1, QK)), _full((1, QK)), _full((QK, QK)),
                     _full((1, QK)), _full((1, QK)), _full((1, QK)),
                     _full((C, QK)), _full((1, C))]),
        out_specs=pl.BlockSpec((TILE, C), lambda i: (i, 0)),
        out_shape=jax.ShapeDtypeStruct((F, C), f32),
    )(kx3, txc, xq, gv3, s1, s2, s3, *ge_args, r32(neg0), r32(nebe0), neW1,
      r32(neb1), r32(neg1), r32(nebe1), neW2, neb2.reshape(1, C))

    return out.reshape(B, N, C).transpose(0, 2, 1)


# speedup vs baseline: 18.9016x; 2.5667x over previous
import functools

import jax
import jax.numpy as jnp
from jax import lax
from jax.experimental import pallas as pl
from jax.experimental.pallas import tpu as pltpu
from jax.experimental.pallas import tpu_sc as plsc

B, C, N, S = 2, 128, 10000, 16
QK = 32
NPAD = 10240
F = B * N
FS = F * S
FP = FS // 4
TILE = 400
T4 = TILE * 4
GRID = F // TILE
TN = 1024
MF = float(FS)
CHUNK = 128
NCHUNK = FS // CHUNK
NW = 32
ITERS = (NCHUNK + NW - 1) // NW

_dims = (((1,), (1,)), ((), ()))


def _pcall(body, **kw):
    return pl.pallas_call(body, **kw)


def _dot(a, b):
    return lax.dot_general(a, b, (((1,), (0,)), ((), ())),
                           preferred_element_type=jnp.float32)


def _proj_body(x_ref, y_ref, xyz_ref, wq_ref, bq_ref, wk_ref, bk_ref,
               wv_ref, bv_ref, xq_ref, tkx_ref, tv_ref):
    xb = x_ref[0]
    yb = y_ref[0]
    d0 = (((0,), (1,)), ((), ()))
    xq_ref[...] = lax.dot_general(xb, wq_ref[...], d0,
                                  preferred_element_type=jnp.float32) + bq_ref[...]
    k = lax.dot_general(yb, wk_ref[...], d0,
                        preferred_element_type=jnp.float32) + bk_ref[...]
    tkx_ref[...] = jnp.concatenate(
        [k, xyz_ref[...], jnp.zeros((TN, 128 - QK - 16), jnp.float32)], axis=1)
    tv_ref[...] = lax.dot_general(yb, wv_ref[...], d0,
                                  preferred_element_type=jnp.float32) + bv_ref[...]


def _projections(x_p, y_p, xyzp, Wq, bq, Wk, bk, Wv, bv):
    nt = NPAD // TN
    return _pcall(
        _proj_body,
        grid=(B, nt),
        in_specs=[
            pl.BlockSpec((1, C, TN), lambda b, i: (b, 0, i)),
            pl.BlockSpec((1, C, TN), lambda b, i: (b, 0, i)),
            pl.BlockSpec((TN, 16), lambda b, i: (b * (NPAD // TN) + i, 0)),
            pl.BlockSpec((QK, C), lambda b, i: (0, 0)),
            pl.BlockSpec((1, QK), lambda b, i: (0, 0)),
            pl.BlockSpec((QK, C), lambda b, i: (0, 0)),
            pl.BlockSpec((1, QK), lambda b, i: (0, 0)),
            pl.BlockSpec((C, C), lambda b, i: (0, 0)),
            pl.BlockSpec((1, C), lambda b, i: (0, 0)),
        ],
        out_specs=[
            pl.BlockSpec((TN, QK), lambda b, i: (b * (NPAD // TN) + i, 0)),
            pl.BlockSpec((TN, C), lambda b, i: (b * (NPAD // TN) + i, 0)),
            pl.BlockSpec((TN, C), lambda b, i: (b * (NPAD // TN) + i, 0)),
        ],
        out_shape=[
            jax.ShapeDtypeStruct((B * NPAD, QK), jnp.float32),
            jax.ShapeDtypeStruct((B * NPAD, C), jnp.float32),
            jax.ShapeDtypeStruct((B * NPAD, C), jnp.float32),
        ],
    )(x_p, y_p, xyzp, Wq, bq, Wk, bk, Wv, bv)


def _sc_gather(gidx, gidxp, tkx, tv):
    mesh = plsc.VectorSubcoreMesh(core_axis_name="c", subcore_axis_name="s")

    @functools.partial(
        pl.kernel,
        mesh=mesh,
        out_type=[
            jax.ShapeDtypeStruct((FP, C), jnp.float32),
            jax.ShapeDtypeStruct((FP, C), jnp.float32),
            jax.ShapeDtypeStruct((FS, C), jnp.float32),
        ],
        scratch_types=[
            pltpu.VMEM((CHUNK,), jnp.int32),
            pltpu.VMEM((CHUNK,), jnp.int32),
            pltpu.VMEM((CHUNK, C), jnp.float32),
            pltpu.VMEM((CHUNK, C), jnp.float32),
            pltpu.VMEM((32, C), jnp.float32),
            pltpu.VMEM((32, C), jnp.float32),
            pltpu.VMEM((8, C), jnp.float32),
            pltpu.SemaphoreType.DMA,
        ],
    )
    def k(gidx_ref, gidxp_ref, tkx_ref, tv_ref, gkp_ref, gu_ref, gv_ref,
          idx_v, idxp_v, kxbuf, vbuf, kpk, ubuf, cbuf, sem):
        wid = lax.axis_index("s") * 2 + lax.axis_index("c")
        zero16 = jnp.zeros((16,), jnp.float32)

        def zbody(r, carry):
            for lg in range(4):
                ubuf[r, pl.ds(lg * 32 + 16, 16)] = zero16
            return carry

        lax.fori_loop(0, 32, zbody, 0)

        def body(j, carry):
            cid = wid + j * NW

            @pl.when(cid < NCHUNK)
            def _():
                base = cid * CHUNK
                pltpu.sync_copy(gidx_ref.at[pl.ds(base, CHUNK)], idx_v)
                pltpu.sync_copy(gidxp_ref.at[pl.ds(base, CHUNK)], idxp_v)
                g1 = pltpu.async_copy(tkx_ref.at[idx_v], kxbuf, sem)
                g2 = pltpu.async_copy(tv_ref.at[idxp_v], vbuf, sem)
                f0 = cid * 8
                b = f0 // N
                pt0 = b * NPAD + (f0 - b * N)
                pltpu.sync_copy(tkx_ref.at[pl.ds(pt0, 8)], cbuf)
                g1.wait()
                g2.wait()

                def pbody(p, carry2):
                    cvec = cbuf[p, pl.ds(QK, 16)]
                    for s in range(16):
                        u = p * 16 + s
                        g = kxbuf[u, pl.ds(QK, 16)]
                        dv = g - cvec
                        r_out = p * 4 + s // 4
                        lg = s % 4
                        ubuf[r_out, pl.ds(lg * 32, 16)] = dv
                        kpk[r_out, pl.ds(lg * 32, 16)] = kxbuf[u, pl.ds(0, 16)]
                        kpk[r_out, pl.ds(lg * 32 + 16, 16)] = \
                            kxbuf[u, pl.ds(16, 16)]
                    return carry2

                lax.fori_loop(0, 8, pbody, 0)
                s1 = pltpu.async_copy(kpk, gkp_ref.at[pl.ds(cid * 32, 32)], sem)
                s2 = pltpu.async_copy(ubuf, gu_ref.at[pl.ds(cid * 32, 32)], sem)
                s3 = pltpu.async_copy(vbuf, gv_ref.at[pl.ds(base, CHUNK)], sem)
                s1.wait()
                s2.wait()
                s3.wait()

            return carry

        lax.fori_loop(0, ITERS, body, 0)

    return k(gidx, gidxp, tkx, tv)


def _euc(u, msum):
    sqv = _dot(u * u, msum)
    lane = lax.broadcasted_iota(jnp.int32, u.shape, 1) % QK
    return jnp.where(lane == 3, jnp.sqrt(sqv + 1e-12), u)


def _acc_stats(s_ref, v):
    @pl.when(pl.program_id(0) == 0)
    def _():
        s_ref[...] = jnp.zeros_like(s_ref)

    s_ref[0:1, :] += jnp.sum(v, axis=0)[None]
    s_ref[1:2, :] += jnp.sum(v * v, axis=0)[None]


def _c1_body(gu_ref, msum_ref, p_ref, su_ref):
    ue = _euc(gu_ref[...], msum_ref[...])
    i = pl.program_id(0)

    @pl.when(i == 0)
    def _():
        p_ref[...] = jnp.zeros_like(p_ref)
        su_ref[...] = jnp.zeros_like(su_ref)

    p_ref[...] += lax.dot_general(ue, ue, (((0,), (0,)), ((), ())),
                                  preferred_element_type=jnp.float32)
    su_ref[0:1, :] += jnp.sum(ue, axis=0)[None]


def _c2_body(gkp_ref, gu_ref, msum_ref, xq_ref, w1_ref, b1_ref, a1_ref,
             c1_ref, w2_ref, b2_ref, emb_ref, s2_ref):
    ue = _euc(gu_ref[...], msum_ref[...])
    hp = _dot(ue, w1_ref[...]) + b1_ref[...]
    h1 = jnp.maximum(hp * a1_ref[...] + c1_ref[...], 0.0)
    h = _dot(h1, w2_ref[...]) + b2_ref[...]
    q = jnp.concatenate([xq_ref[...]] * 4, axis=1)
    q4 = jnp.broadcast_to(q[:, None, :], (TILE, 4, C)).reshape(T4, C)
    d = gkp_ref[...] - q4
    emb = h - d * d
    emb_ref[...] = emb
    _acc_stats(s2_ref, emb)


def _c3_body(emb_ref, a2_ref, c2_ref, w1_ref, b1_ref, s3_ref):
    r = jnp.maximum(emb_ref[...] * a2_ref[...] + c2_ref[...], 0.0)
    t1 = _dot(r, w1_ref[...]) + b1_ref[...]
    _acc_stats(s3_ref, t1)


def _c4_body(emb_ref, gv0_ref, gv1_ref, gv2_ref, gv3_ref, a2_ref, c2_ref,
             w1_ref, b1_ref, a3_ref, c3_ref, w2_ref, b2_ref, out_ref):
    r = jnp.maximum(emb_ref[...] * a2_ref[...] + c2_ref[...], 0.0)
    t1 = _dot(r, w1_ref[...]) + b1_ref[...]
    t1 = jnp.maximum(t1 * a3_ref[...] + c3_ref[...], 0.0)
    t2 = _dot(t1, w2_ref[...]) + b2_ref[...]
    ts = [t2[:, j * C:(j + 1) * C] for j in range(4)]
    m = jnp.maximum(jnp.maximum(ts[0], ts[1]), jnp.maximum(ts[2], ts[3]))
    m = jnp.max(m.reshape(TILE, 4, C), axis=1)
    mr = jnp.broadcast_to(m[:, None, :], (TILE, 4, C)).reshape(T4, C)
    es = [jnp.exp(t - mr) for t in ts]
    den = jnp.sum((es[0] + es[1] + es[2] + es[3]).reshape(TILE, 4, C), axis=1)
    acc = (es[0] * gv0_ref[0] + es[1] * gv1_ref[0]
           + es[2] * gv2_ref[0] + es[3] * gv3_ref[0])
    out_ref[...] = jnp.sum(acc.reshape(TILE, 4, C), axis=1) / den


def _full(shape):
    return pl.BlockSpec(shape, lambda i: tuple(0 for _ in shape))


def _fold(row):
    return row.reshape(4, QK).sum(axis=0)


def _tile4(v):
    return jnp.tile(v.reshape(1, QK), (1, 4))


def kernel(x, y, y_xyz, idx, Wq, bq, Wk, bk, Wv, bv, geW1, geb1, geg1, gebe1,
           geW2, geb2, neg0, nebe0, neW1, neb1, neg1, nebe1, neW2, neb2):
    f32 = jnp.float32
    x_p = jnp.pad(x, ((0, 0), (0, 0), (0, NPAD - N)))
    y_p = jnp.pad(y, ((0, 0), (0, 0), (0, NPAD - N)))

    xyzt = jnp.transpose(y_xyz, (0, 2, 1))
    txp = jnp.pad(xyzt, ((0, 0), (0, NPAD - N), (0, 13))).reshape(B * NPAD, 16)

    xqp, tkx, tv = _projections(
        x_p, y_p, txp, Wq, bq.reshape(1, QK).astype(f32), Wk,
        bk.reshape(1, QK).astype(f32), Wv, bv.reshape(1, C).astype(f32))
    xq = xqp.reshape(B, NPAD, QK)[:, :N].reshape(F, QK)

    gidx = (idx.astype(jnp.int32)
            + (jnp.arange(B, dtype=jnp.int32) * NPAD)[:, None, None]
            ).reshape(FS)
    gidxp = gidx.reshape(FP, 4).T.reshape(FS)

    gkp, gu, gv = _sc_gather(gidx, gidxp, tkx, tv)
    gv4 = gv.reshape(4, FP, C)

    eye4 = jnp.eye(4, dtype=f32)
    w1blk = jnp.kron(eye4, jnp.pad(geW1.T, ((0, QK - 4), (0, 0))))
    w2blk = jnp.kron(eye4, geW2.T)
    nw1blk = jnp.kron(eye4, neW1.T)
    w2cat = jnp.concatenate(
        [jnp.pad(neW2.T, ((QK * j, QK * (3 - j)), (0, 0))) for j in range(4)],
        axis=1)
    mblk = jnp.zeros((QK, QK), f32).at[0:3, 3].set(1.0)
    msum = jnp.kron(eye4, mblk)
    nb2cat = jnp.tile(neb2.reshape(1, C), (1, 4))

    gu_spec = pl.BlockSpec((T4, C), lambda i: (i, 0))
    stats_spec = pl.BlockSpec((8, C), lambda i: (0, 0))
    stats_shape = jax.ShapeDtypeStruct((8, C), f32)

    p_mat, su = _pcall(
        _c1_body,
        grid=(GRID,),
        in_specs=[gu_spec, _full((C, C))],
        out_specs=[pl.BlockSpec((C, C), lambda i: (0, 0)), stats_spec],
        out_shape=[jax.ShapeDtypeStruct((C, C), f32), stats_shape],
    )(gu, msum)

    eu = _fold(su[0])[:4] / MF
    p4 = jnp.einsum('aiaj->ij', p_mat.reshape(4, QK, 4, QK))[:4, :4] / MF
    cov = p4 - jnp.outer(eu, eu)
    mu1 = geW1 @ eu + geb1
    var1 = jnp.einsum('oi,ij,oj->o', geW1, cov, geW1)
    a1 = geg1 * lax.rsqrt(var1 + 1e-5)
    c1 = gebe1 - mu1 * a1

    emb, s2 = _pcall(
        _c2_body,
        grid=(GRID,),
        in_specs=[gu_spec, gu_spec, _full((C, C)),
                  pl.BlockSpec((TILE, QK), lambda i: (i, 0)),
                  _full((C, C)), _full((1, C)), _full((1, C)), _full((1, C)),
                  _full((C, C)), _full((1, C))],
        out_specs=[gu_spec, stats_spec],
        out_shape=[jax.ShapeDtypeStruct((FP, C), f32), stats_shape],
    )(gkp, gu, msum, xq, w1blk, _tile4(geb1), _tile4(a1), _tile4(c1), w2blk,
      _tile4(geb2))

    mu2 = _fold(s2[0]) / MF
    var2 = _fold(s2[1]) / MF - mu2 * mu2
    a2 = neg0 * lax.rsqrt(var2 + 1e-5)
    c2 = nebe0 - mu2 * a2

    s3 = _pcall(
        _c3_body,
        grid=(GRID,),
        in_specs=[gu_spec, _full((1, C)), _full((1, C)), _full((C, C)),
                  _full((1, C))],
        out_specs=stats_spec,
        out_shape=stats_shape,
    )(emb, _tile4(a2), _tile4(c2), nw1blk, _tile4(neb1))

    mu3 = _fold(s3[0]) / MF
    var3 = _fold(s3[1]) / MF - mu3 * mu3
    a3 = neg1 * lax.rsqrt(var3 + 1e-5)
    c3 = nebe1 - mu3 * a3

    out = _pcall(
        _c4_body,
        grid=(GRID,),
        in_specs=([gu_spec]
                  + [pl.BlockSpec((1, T4, C), lambda i, j=j: (j, i, 0))
                     for j in range(4)]
                  + [_full((1, C)), _full((1, C)), _full((C, C)),
                     _full((1, C)), _full((1, C)), _full((1, C)),
                     _full((C, 4 * C)), _full((1, 4 * C))]),
        out_specs=pl.BlockSpec((TILE, C), lambda i: (i, 0)),
        out_shape=jax.ShapeDtypeStruct((F, C), f32),
    )(emb, gv4, gv4, gv4, gv4, _tile4(a2), _tile4(c2), nw1blk, _tile4(neb1),
      _tile4(a3), _tile4(c3), w2cat, nb2cat)

    return out.reshape(B, N, C).transpose(0, 2, 1)


# speedup vs baseline: 21.7371x; 1.1500x over previous
import functools

import jax
import jax.numpy as jnp
from jax import lax
from jax.experimental import pallas as pl
from jax.experimental.pallas import tpu as pltpu
from jax.experimental.pallas import tpu_sc as plsc

B, C, N, S = 2, 128, 10000, 16
QK = 32
NPAD = 10240
F = B * N
FS = F * S
FP = FS // 4
TILE = 400
T4 = TILE * 4
GRID = F // TILE
TN = 1024
MF = float(FS)
CHUNK = 128
NCHUNK = FS // CHUNK
NW = 32
ITERS = (NCHUNK + NW - 1) // NW

_dims = (((1,), (1,)), ((), ()))


def _pcall(body, **kw):
    return pl.pallas_call(body, **kw)


def _dot(a, b):
    return lax.dot_general(a, b, (((1,), (0,)), ((), ())),
                           preferred_element_type=jnp.float32)


def _proj_body(x_ref, y_ref, xyz_ref, wq_ref, bq_ref, wk_ref, bk_ref,
               wv_ref, bv_ref, xq_ref, tkx_ref, tv_ref):
    xb = x_ref[0]
    yb = y_ref[0]
    d0 = (((0,), (1,)), ((), ()))
    xq_ref[...] = lax.dot_general(xb, wq_ref[...], d0,
                                  preferred_element_type=jnp.float32) + bq_ref[...]
    k = lax.dot_general(yb, wk_ref[...], d0,
                        preferred_element_type=jnp.float32) + bk_ref[...]
    tkx_ref[...] = jnp.concatenate(
        [k, xyz_ref[...], jnp.zeros((TN, 128 - QK - 16), jnp.float32)], axis=1)
    tv_ref[...] = lax.dot_general(yb, wv_ref[...], d0,
                                  preferred_element_type=jnp.float32) + bv_ref[...]


def _projections(x_p, y_p, xyzp, Wq, bq, Wk, bk, Wv, bv):
    nt = NPAD // TN
    return _pcall(
        _proj_body,
        grid=(B, nt),
        in_specs=[
            pl.BlockSpec((1, C, TN), lambda b, i: (b, 0, i)),
            pl.BlockSpec((1, C, TN), lambda b, i: (b, 0, i)),
            pl.BlockSpec((TN, 16), lambda b, i: (b * (NPAD // TN) + i, 0)),
            pl.BlockSpec((QK, C), lambda b, i: (0, 0)),
            pl.BlockSpec((1, QK), lambda b, i: (0, 0)),
            pl.BlockSpec((QK, C), lambda b, i: (0, 0)),
            pl.BlockSpec((1, QK), lambda b, i: (0, 0)),
            pl.BlockSpec((C, C), lambda b, i: (0, 0)),
            pl.BlockSpec((1, C), lambda b, i: (0, 0)),
        ],
        out_specs=[
            pl.BlockSpec((TN, QK), lambda b, i: (b * (NPAD // TN) + i, 0)),
            pl.BlockSpec((TN, C), lambda b, i: (b * (NPAD // TN) + i, 0)),
            pl.BlockSpec((TN, C), lambda b, i: (b * (NPAD // TN) + i, 0)),
        ],
        out_shape=[
            jax.ShapeDtypeStruct((B * NPAD, QK), jnp.float32),
            jax.ShapeDtypeStruct((B * NPAD, C), jnp.float32),
            jax.ShapeDtypeStruct((B * NPAD, C), jnp.float32),
        ],
    )(x_p, y_p, xyzp, Wq, bq, Wk, bk, Wv, bv)


def _sc_gather_kx(gidx, tkx):
    mesh = plsc.VectorSubcoreMesh(core_axis_name="c", subcore_axis_name="s")

    @functools.partial(
        pl.kernel,
        mesh=mesh,
        out_type=[
            jax.ShapeDtypeStruct((FP, C), jnp.float32),
            jax.ShapeDtypeStruct((FP, C), jnp.float32),
        ],
        scratch_types=[
            pltpu.VMEM((CHUNK,), jnp.int32),
            pltpu.VMEM((CHUNK, C), jnp.float32),
            pltpu.VMEM((32, C), jnp.float32),
            pltpu.VMEM((32, C), jnp.float32),
            pltpu.VMEM((8, C), jnp.float32),
            pltpu.SemaphoreType.DMA,
        ],
    )
    def k(gidx_ref, tkx_ref, gkp_ref, gu_ref,
          idx_v, kxbuf, kpk, ubuf, cbuf, sem):
        wid = lax.axis_index("s") * 2 + lax.axis_index("c")
        zero16 = jnp.zeros((16,), jnp.float32)

        def zbody(r, carry):
            for lg in range(4):
                ubuf[r, pl.ds(lg * 32 + 16, 16)] = zero16
            return carry

        lax.fori_loop(0, 32, zbody, 0)

        def body(j, carry):
            cid = wid + j * NW

            @pl.when(cid < NCHUNK)
            def _():
                base = cid * CHUNK
                pltpu.sync_copy(gidx_ref.at[pl.ds(base, CHUNK)], idx_v)
                g1 = pltpu.async_copy(tkx_ref.at[idx_v], kxbuf, sem)
                f0 = cid * 8
                b = f0 // N
                pt0 = b * NPAD + (f0 - b * N)
                pltpu.sync_copy(tkx_ref.at[pl.ds(pt0, 8)], cbuf)
                g1.wait()

                def pbody(p, carry2):
                    cvec = cbuf[p, pl.ds(QK, 16)]
                    for s in range(16):
                        u = p * 16 + s
                        g = kxbuf[u, pl.ds(QK, 16)]
                        dv = g - cvec
                        r_out = p * 4 + s // 4
                        lg = s % 4
                        ubuf[r_out, pl.ds(lg * 32, 16)] = dv
                        kpk[r_out, pl.ds(lg * 32, 16)] = kxbuf[u, pl.ds(0, 16)]
                        kpk[r_out, pl.ds(lg * 32 + 16, 16)] = \
                            kxbuf[u, pl.ds(16, 16)]
                    return carry2

                lax.fori_loop(0, 8, pbody, 0)
                s1 = pltpu.async_copy(kpk, gkp_ref.at[pl.ds(cid * 32, 32)], sem)
                s2 = pltpu.async_copy(ubuf, gu_ref.at[pl.ds(cid * 32, 32)], sem)
                s1.wait()
                s2.wait()

            return carry

        lax.fori_loop(0, ITERS, body, 0)

    return k(gidx, tkx)


def _sc_gather_v(gidxp, tv):
    mesh = plsc.VectorSubcoreMesh(core_axis_name="c", subcore_axis_name="s")

    @functools.partial(
        pl.kernel,
        mesh=mesh,
        out_type=jax.ShapeDtypeStruct((FS, C), jnp.float32),
        scratch_types=[
            pltpu.VMEM((CHUNK,), jnp.int32),
            pltpu.VMEM((CHUNK, C), jnp.float32),
            pltpu.SemaphoreType.DMA,
        ],
    )
    def k(gidxp_ref, tv_ref, gv_ref, idxp_v, vbuf, sem):
        wid = lax.axis_index("s") * 2 + lax.axis_index("c")

        def body(j, carry):
            cid = wid + j * NW

            @pl.when(cid < NCHUNK)
            def _():
                base = cid * CHUNK
                pltpu.sync_copy(gidxp_ref.at[pl.ds(base, CHUNK)], idxp_v)
                pltpu.async_copy(tv_ref.at[idxp_v], vbuf, sem).wait()
                pltpu.async_copy(vbuf, gv_ref.at[pl.ds(base, CHUNK)],
                                 sem).wait()

            return carry

        lax.fori_loop(0, ITERS, body, 0)

    return k(gidxp, tv)


def _euc(u, msum):
    sqv = _dot(u * u, msum)
    lane = lax.broadcasted_iota(jnp.int32, u.shape, 1) % QK
    return jnp.where(lane == 3, jnp.sqrt(sqv + 1e-12), u)


def _acc_stats(s_ref, v):
    @pl.when(pl.program_id(0) == 0)
    def _():
        s_ref[...] = jnp.zeros_like(s_ref)

    s_ref[0:1, :] += jnp.sum(v, axis=0)[None]
    s_ref[1:2, :] += jnp.sum(v * v, axis=0)[None]


def _c1_body(gu_ref, msum_ref, p_ref, su_ref):
    ue = _euc(gu_ref[...], msum_ref[...])
    i = pl.program_id(0)

    @pl.when(i == 0)
    def _():
        p_ref[...] = jnp.zeros_like(p_ref)
        su_ref[...] = jnp.zeros_like(su_ref)

    p_ref[...] += lax.dot_general(ue, ue, (((0,), (0,)), ((), ())),
                                  preferred_element_type=jnp.float32)
    su_ref[0:1, :] += jnp.sum(ue, axis=0)[None]


def _c2_body(gkp_ref, gu_ref, msum_ref, xq_ref, w1_ref, b1_ref, a1_ref,
             c1_ref, w2_ref, b2_ref, emb_ref, s2_ref):
    ue = _euc(gu_ref[...], msum_ref[...])
    hp = _dot(ue, w1_ref[...]) + b1_ref[...]
    h1 = jnp.maximum(hp * a1_ref[...] + c1_ref[...], 0.0)
    h = _dot(h1, w2_ref[...]) + b2_ref[...]
    q = jnp.concatenate([xq_ref[...]] * 4, axis=1)
    q4 = jnp.broadcast_to(q[:, None, :], (TILE, 4, C)).reshape(T4, C)
    d = gkp_ref[...] - q4
    emb = h - d * d
    emb_ref[...] = emb
    _acc_stats(s2_ref, emb)


def _c3_body(emb_ref, a2_ref, c2_ref, w1_ref, b1_ref, s3_ref):
    r = jnp.maximum(emb_ref[...] * a2_ref[...] + c2_ref[...], 0.0)
    t1 = _dot(r, w1_ref[...]) + b1_ref[...]
    _acc_stats(s3_ref, t1)


def _c4_body(emb_ref, gv0_ref, gv1_ref, gv2_ref, gv3_ref, a2_ref, c2_ref,
             w1_ref, b1_ref, a3_ref, c3_ref, w2_ref, b2_ref, out_ref):
    r = jnp.maximum(emb_ref[...] * a2_ref[...] + c2_ref[...], 0.0)
    t1 = _dot(r, w1_ref[...]) + b1_ref[...]
    t1 = jnp.maximum(t1 * a3_ref[...] + c3_ref[...], 0.0)
    t2 = _dot(t1, w2_ref[...]) + b2_ref[...]
    ts = [t2[:, j * C:(j + 1) * C] for j in range(4)]
    m = jnp.maximum(jnp.maximum(ts[0], ts[1]), jnp.maximum(ts[2], ts[3]))
    m = jnp.max(m.reshape(TILE, 4, C), axis=1)
    mr = jnp.broadcast_to(m[:, None, :], (TILE, 4, C)).reshape(T4, C)
    es = [jnp.exp(t - mr) for t in ts]
    den = jnp.sum((es[0] + es[1] + es[2] + es[3]).reshape(TILE, 4, C), axis=1)
    acc = (es[0] * gv0_ref[0] + es[1] * gv1_ref[0]
           + es[2] * gv2_ref[0] + es[3] * gv3_ref[0])
    out_ref[...] = jnp.sum(acc.reshape(TILE, 4, C), axis=1) / den


def _full(shape):
    return pl.BlockSpec(shape, lambda i: tuple(0 for _ in shape))


def _fold(row):
    return row.reshape(4, QK).sum(axis=0)


def _tile4(v):
    return jnp.tile(v.reshape(1, QK), (1, 4))


def kernel(x, y, y_xyz, idx, Wq, bq, Wk, bk, Wv, bv, geW1, geb1, geg1, gebe1,
           geW2, geb2, neg0, nebe0, neW1, neb1, neg1, nebe1, neW2, neb2):
    f32 = jnp.float32
    x_p = jnp.pad(x, ((0, 0), (0, 0), (0, NPAD - N)))
    y_p = jnp.pad(y, ((0, 0), (0, 0), (0, NPAD - N)))

    xyzt = jnp.transpose(y_xyz, (0, 2, 1))
    txp = jnp.pad(xyzt, ((0, 0), (0, NPAD - N), (0, 13))).reshape(B * NPAD, 16)

    xqp, tkx, tv = _projections(
        x_p, y_p, txp, Wq, bq.reshape(1, QK).astype(f32), Wk,
        bk.reshape(1, QK).astype(f32), Wv, bv.reshape(1, C).astype(f32))
    xq = xqp.reshape(B, NPAD, QK)[:, :N].reshape(F, QK)

    gidx = (idx.astype(jnp.int32)
            + (jnp.arange(B, dtype=jnp.int32) * NPAD)[:, None, None]
            ).reshape(FS)
    gidxp = gidx.reshape(FP, 4).T.reshape(FS)

    gkp, gu = _sc_gather_kx(gidx, tkx)
    gv = _sc_gather_v(gidxp, tv)
    gv4 = gv.reshape(4, FP, C)

    eye4 = jnp.eye(4, dtype=f32)
    w1blk = jnp.kron(eye4, jnp.pad(geW1.T, ((0, QK - 4), (0, 0))))
    w2blk = jnp.kron(eye4, geW2.T)
    nw1blk = jnp.kron(eye4, neW1.T)
    w2cat = jnp.concatenate(
        [jnp.pad(neW2.T, ((QK * j, QK * (3 - j)), (0, 0))) for j in range(4)],
        axis=1)
    mblk = jnp.zeros((QK, QK), f32).at[0:3, 3].set(1.0)
    msum = jnp.kron(eye4, mblk)
    nb2cat = jnp.tile(neb2.reshape(1, C), (1, 4))

    gu_spec = pl.BlockSpec((T4, C), lambda i: (i, 0))
    stats_spec = pl.BlockSpec((8, C), lambda i: (0, 0))
    stats_shape = jax.ShapeDtypeStruct((8, C), f32)

    p_mat, su = _pcall(
        _c1_body,
        grid=(GRID,),
        in_specs=[gu_spec, _full((C, C))],
        out_specs=[pl.BlockSpec((C, C), lambda i: (0, 0)), stats_spec],
        out_shape=[jax.ShapeDtypeStruct((C, C), f32), stats_shape],
    )(gu, msum)

    eu = _fold(su[0])[:4] / MF
    p4 = jnp.einsum('aiaj->ij', p_mat.reshape(4, QK, 4, QK))[:4, :4] / MF
    cov = p4 - jnp.outer(eu, eu)
    mu1 = geW1 @ eu + geb1
    var1 = jnp.einsum('oi,ij,oj->o', geW1, cov, geW1)
    a1 = geg1 * lax.rsqrt(var1 + 1e-5)
    c1 = gebe1 - mu1 * a1

    emb, s2 = _pcall(
        _c2_body,
        grid=(GRID,),
        in_specs=[gu_spec, gu_spec, _full((C, C)),
                  pl.BlockSpec((TILE, QK), lambda i: (i, 0)),
                  _full((C, C)), _full((1, C)), _full((1, C)), _full((1, C)),
                  _full((C, C)), _full((1, C))],
        out_specs=[gu_spec, stats_spec],
        out_shape=[jax.ShapeDtypeStruct((FP, C), f32), stats_shape],
    )(gkp, gu, msum, xq, w1blk, _tile4(geb1), _tile4(a1), _tile4(c1), w2blk,
      _tile4(geb2))

    mu2 = _fold(s2[0]) / MF
    var2 = _fold(s2[1]) / MF - mu2 * mu2
    a2 = neg0 * lax.rsqrt(var2 + 1e-5)
    c2 = nebe0 - mu2 * a2

    s3 = _pcall(
        _c3_body,
        grid=(GRID,),
        in_specs=[gu_spec, _full((1, C)), _full((1, C)), _full((C, C)),
                  _full((1, C))],
        out_specs=stats_spec,
        out_shape=stats_shape,
    )(emb, _tile4(a2), _tile4(c2), nw1blk, _tile4(neb1))

    mu3 = _fold(s3[0]) / MF
    var3 = _fold(s3[1]) / MF - mu3 * mu3
    a3 = neg1 * lax.rsqrt(var3 + 1e-5)
    c3 = nebe1 - mu3 * a3

    out = _pcall(
        _c4_body,
        grid=(GRID,),
        in_specs=([gu_spec]
                  + [pl.BlockSpec((1, T4, C), lambda i, j=j: (j, i, 0))
                     for j in range(4)]
                  + [_full((1, C)), _full((1, C)), _full((C, C)),
                     _full((1, C)), _full((1, C)), _full((1, C)),
                     _full((C, 4 * C)), _full((1, 4 * C))]),
        out_specs=pl.BlockSpec((TILE, C), lambda i: (i, 0)),
        out_shape=jax.ShapeDtypeStruct((F, C), f32),
    )(emb, gv4, gv4, gv4, gv4, _tile4(a2), _tile4(c2), nw1blk, _tile4(neb1),
      _tile4(a3), _tile4(c3), w2cat, nb2cat)

    return out.reshape(B, N, C).transpose(0, 2, 1)


# speedup vs baseline: 22.3850x; 1.0298x over previous
import functools

import jax
import jax.numpy as jnp
from jax import lax
from jax.experimental import pallas as pl
from jax.experimental.pallas import tpu as pltpu
from jax.experimental.pallas import tpu_sc as plsc

B, C, N, S = 2, 128, 10000, 16
QK = 32
NPAD = 10240
F = B * N
FS = F * S
FP = FS // 4
TILE = 400
T4 = TILE * 4
GRID = F // TILE
TN = 1024
MF = float(FS)
KXW = QK + 16
CHUNK = 128
NCHUNK = FS // CHUNK
NW = 32
ITERS = (NCHUNK + NW - 1) // NW

_dims = (((1,), (1,)), ((), ()))


def _pcall(body, **kw):
    return pl.pallas_call(body, **kw)


def _dot(a, b):
    return lax.dot_general(a, b, (((1,), (0,)), ((), ())),
                           preferred_element_type=jnp.float32)


def _proj_body(x_ref, y_ref, xyz_ref, wq_ref, bq_ref, wk_ref, bk_ref,
               wv_ref, bv_ref, xq_ref, tkx_ref, tv_ref):
    xb = x_ref[0]
    yb = y_ref[0]
    d0 = (((0,), (1,)), ((), ()))
    xq_ref[...] = lax.dot_general(xb, wq_ref[...], d0,
                                  preferred_element_type=jnp.float32) + bq_ref[...]
    k = lax.dot_general(yb, wk_ref[...], d0,
                        preferred_element_type=jnp.float32) + bk_ref[...]
    tkx_ref[...] = jnp.concatenate([k, xyz_ref[...]], axis=1)
    tv_ref[...] = lax.dot_general(yb, wv_ref[...], d0,
                                  preferred_element_type=jnp.float32) + bv_ref[...]


def _projections(x_p, y_p, xyzp, Wq, bq, Wk, bk, Wv, bv):
    nt = NPAD // TN
    return _pcall(
        _proj_body,
        grid=(B, nt),
        in_specs=[
            pl.BlockSpec((1, C, TN), lambda b, i: (b, 0, i)),
            pl.BlockSpec((1, C, TN), lambda b, i: (b, 0, i)),
            pl.BlockSpec((TN, 16), lambda b, i: (b * (NPAD // TN) + i, 0)),
            pl.BlockSpec((QK, C), lambda b, i: (0, 0)),
            pl.BlockSpec((1, QK), lambda b, i: (0, 0)),
            pl.BlockSpec((QK, C), lambda b, i: (0, 0)),
            pl.BlockSpec((1, QK), lambda b, i: (0, 0)),
            pl.BlockSpec((C, C), lambda b, i: (0, 0)),
            pl.BlockSpec((1, C), lambda b, i: (0, 0)),
        ],
        out_specs=[
            pl.BlockSpec((TN, QK), lambda b, i: (b * (NPAD // TN) + i, 0)),
            pl.BlockSpec((TN, KXW), lambda b, i: (b * (NPAD // TN) + i, 0)),
            pl.BlockSpec((TN, C), lambda b, i: (b * (NPAD // TN) + i, 0)),
        ],
        out_shape=[
            jax.ShapeDtypeStruct((B * NPAD, QK), jnp.float32),
            jax.ShapeDtypeStruct((B * NPAD, KXW), jnp.float32),
            jax.ShapeDtypeStruct((B * NPAD, C), jnp.float32),
        ],
    )(x_p, y_p, xyzp, Wq, bq, Wk, bk, Wv, bv)


def _sc_gather_kx(gidx, tkx):
    mesh = plsc.VectorSubcoreMesh(core_axis_name="c", subcore_axis_name="s")

    @functools.partial(
        pl.kernel,
        mesh=mesh,
        out_type=[
            jax.ShapeDtypeStruct((FP, C), jnp.float32),
            jax.ShapeDtypeStruct((FP, C), jnp.float32),
        ],
        scratch_types=[
            pltpu.VMEM((CHUNK,), jnp.int32),
            pltpu.VMEM((CHUNK, KXW), jnp.float32),
            pltpu.VMEM((32, C), jnp.float32),
            pltpu.VMEM((32, C), jnp.float32),
            pltpu.VMEM((8, KXW), jnp.float32),
            pltpu.SemaphoreType.DMA,
        ],
        compiler_params=pltpu.CompilerParams(use_tc_tiling_on_sc=False),
    )
    def k(gidx_ref, tkx_ref, gkp_ref, gu_ref,
          idx_v, kxbuf, kpk, ubuf, cbuf, sem):
        wid = lax.axis_index("s") * 2 + lax.axis_index("c")
        zero16 = jnp.zeros((16,), jnp.float32)

        def zbody(r, carry):
            for lg in range(4):
                ubuf[r, pl.ds(lg * 32 + 16, 16)] = zero16
            return carry

        lax.fori_loop(0, 32, zbody, 0)

        def body(j, carry):
            cid = wid + j * NW

            @pl.when(cid < NCHUNK)
            def _():
                base = cid * CHUNK
                pltpu.sync_copy(gidx_ref.at[pl.ds(base, CHUNK)], idx_v)
                g1 = pltpu.async_copy(tkx_ref.at[idx_v], kxbuf, sem)
                f0 = cid * 8
                b = f0 // N
                pt0 = b * NPAD + (f0 - b * N)
                pltpu.sync_copy(tkx_ref.at[pl.ds(pt0, 8)], cbuf)
                g1.wait()

                def pbody(p, carry2):
                    cvec = cbuf[p, pl.ds(QK, 16)]
                    for s in range(16):
                        u = p * 16 + s
                        g = kxbuf[u, pl.ds(QK, 16)]
                        dv = g - cvec
                        r_out = p * 4 + s // 4
                        lg = s % 4
                        ubuf[r_out, pl.ds(lg * 32, 16)] = dv
                        kpk[r_out, pl.ds(lg * 32, 16)] = kxbuf[u, pl.ds(0, 16)]
                        kpk[r_out, pl.ds(lg * 32 + 16, 16)] = \
                            kxbuf[u, pl.ds(16, 16)]
                    return carry2

                lax.fori_loop(0, 8, pbody, 0)
                s1 = pltpu.async_copy(kpk, gkp_ref.at[pl.ds(cid * 32, 32)], sem)
                s2 = pltpu.async_copy(ubuf, gu_ref.at[pl.ds(cid * 32, 32)], sem)
                s1.wait()
                s2.wait()

            return carry

        lax.fori_loop(0, ITERS, body, 0)

    return k(gidx, tkx)


def _sc_gather_v(gidxp, tv):
    mesh = plsc.VectorSubcoreMesh(core_axis_name="c", subcore_axis_name="s")

    @functools.partial(
        pl.kernel,
        mesh=mesh,
        out_type=jax.ShapeDtypeStruct((FS, C), jnp.float32),
        scratch_types=[
            pltpu.VMEM((CHUNK,), jnp.int32),
            pltpu.VMEM((CHUNK, C), jnp.float32),
            pltpu.SemaphoreType.DMA,
        ],
    )
    def k(gidxp_ref, tv_ref, gv_ref, idxp_v, vbuf, sem):
        wid = lax.axis_index("s") * 2 + lax.axis_index("c")

        def body(j, carry):
            cid = wid + j * NW

            @pl.when(cid < NCHUNK)
            def _():
                base = cid * CHUNK
                pltpu.sync_copy(gidxp_ref.at[pl.ds(base, CHUNK)], idxp_v)
                pltpu.async_copy(tv_ref.at[idxp_v], vbuf, sem).wait()
                pltpu.async_copy(vbuf, gv_ref.at[pl.ds(base, CHUNK)],
                                 sem).wait()

            return carry

        lax.fori_loop(0, ITERS, body, 0)

    return k(gidxp, tv)


def _euc(u, msum):
    sqv = _dot(u * u, msum)
    lane = lax.broadcasted_iota(jnp.int32, u.shape, 1) % QK
    return jnp.where(lane == 3, jnp.sqrt(sqv + 1e-12), u)


def _acc_stats(s_ref, v):
    @pl.when(pl.program_id(0) == 0)
    def _():
        s_ref[...] = jnp.zeros_like(s_ref)

    s_ref[0:1, :] += jnp.sum(v, axis=0)[None]
    s_ref[1:2, :] += jnp.sum(v * v, axis=0)[None]


def _c1_body(gu_ref, msum_ref, p_ref, su_ref):
    ue = _euc(gu_ref[...], msum_ref[...])
    i = pl.program_id(0)

    @pl.when(i == 0)
    def _():
        p_ref[...] = jnp.zeros_like(p_ref)
        su_ref[...] = jnp.zeros_like(su_ref)

    p_ref[...] += lax.dot_general(ue, ue, (((0,), (0,)), ((), ())),
                                  preferred_element_type=jnp.float32)
    su_ref[0:1, :] += jnp.sum(ue, axis=0)[None]


def _c2_body(gkp_ref, gu_ref, msum_ref, xq_ref, w1_ref, b1_ref, a1_ref,
             c1_ref, w2_ref, b2_ref, emb_ref, s2_ref):
    ue = _euc(gu_ref[...], msum_ref[...])
    hp = _dot(ue, w1_ref[...]) + b1_ref[...]
    h1 = jnp.maximum(hp * a1_ref[...] + c1_ref[...], 0.0)
    h = _dot(h1, w2_ref[...]) + b2_ref[...]
    q = jnp.concatenate([xq_ref[...]] * 4, axis=1)
    q4 = jnp.broadcast_to(q[:, None, :], (TILE, 4, C)).reshape(T4, C)
    d = gkp_ref[...] - q4
    emb = h - d * d
    emb_ref[...] = emb
    _acc_stats(s2_ref, emb)


def _c3_body(emb_ref, a2_ref, c2_ref, w1_ref, b1_ref, s3_ref):
    r = jnp.maximum(emb_ref[...] * a2_ref[...] + c2_ref[...], 0.0)
    t1 = _dot(r, w1_ref[...]) + b1_ref[...]
    _acc_stats(s3_ref, t1)


def _c4_body(emb_ref, gv0_ref, gv1_ref, gv2_ref, gv3_ref, a2_ref, c2_ref,
             w1_ref, b1_ref, a3_ref, c3_ref, w2_ref, b2_ref, out_ref):
    r = jnp.maximum(emb_ref[...] * a2_ref[...] + c2_ref[...], 0.0)
    t1 = _dot(r, w1_ref[...]) + b1_ref[...]
    t1 = jnp.maximum(t1 * a3_ref[...] + c3_ref[...], 0.0)
    t2 = _dot(t1, w2_ref[...]) + b2_ref[...]
    ts = [t2[:, j * C:(j + 1) * C] for j in range(4)]
    m = jnp.maximum(jnp.maximum(ts[0], ts[1]), jnp.maximum(ts[2], ts[3]))
    m = jnp.max(m.reshape(TILE, 4, C), axis=1)
    mr = jnp.broadcast_to(m[:, None, :], (TILE, 4, C)).reshape(T4, C)
    es = [jnp.exp(t - mr) for t in ts]
    den = jnp.sum((es[0] + es[1] + es[2] + es[3]).reshape(TILE, 4, C), axis=1)
    acc = (es[0] * gv0_ref[0] + es[1] * gv1_ref[0]
           + es[2] * gv2_ref[0] + es[3] * gv3_ref[0])
    out_ref[...] = jnp.sum(acc.reshape(TILE, 4, C), axis=1) / den


def _full(shape):
    return pl.BlockSpec(shape, lambda i: tuple(0 for _ in shape))


def _fold(row):
    return row.reshape(4, QK).sum(axis=0)


def _tile4(v):
    return jnp.tile(v.reshape(1, QK), (1, 4))


def kernel(x, y, y_xyz, idx, Wq, bq, Wk, bk, Wv, bv, geW1, geb1, geg1, gebe1,
           geW2, geb2, neg0, nebe0, neW1, neb1, neg1, nebe1, neW2, neb2):
    f32 = jnp.float32
    x_p = jnp.pad(x, ((0, 0), (0, 0), (0, NPAD - N)))
    y_p = jnp.pad(y, ((0, 0), (0, 0), (0, NPAD - N)))

    xyzt = jnp.transpose(y_xyz, (0, 2, 1))
    txp = jnp.pad(xyzt, ((0, 0), (0, NPAD - N), (0, 13))).reshape(B * NPAD, 16)

    xqp, tkx, tv = _projections(
        x_p, y_p, txp, Wq, bq.reshape(1, QK).astype(f32), Wk,
        bk.reshape(1, QK).astype(f32), Wv, bv.reshape(1, C).astype(f32))
    xq = xqp.reshape(B, NPAD, QK)[:, :N].reshape(F, QK)

    gidx = (idx.astype(jnp.int32)
            + (jnp.arange(B, dtype=jnp.int32) * NPAD)[:, None, None]
            ).reshape(FS)
    gidxp = gidx.reshape(FP, 4).T.reshape(FS)

    gkp, gu = _sc_gather_kx(gidx, tkx)
    gv = _sc_gather_v(gidxp, tv)
    gv4 = gv.reshape(4, FP, C)

    eye4 = jnp.eye(4, dtype=f32)
    w1blk = jnp.kron(eye4, jnp.pad(geW1.T, ((0, QK - 4), (0, 0))))
    w2blk = jnp.kron(eye4, geW2.T)
    nw1blk = jnp.kron(eye4, neW1.T)
    w2cat = jnp.concatenate(
        [jnp.pad(neW2.T, ((QK * j, QK * (3 - j)), (0, 0))) for j in range(4)],
        axis=1)
    mblk = jnp.zeros((QK, QK), f32).at[0:3, 3].set(1.0)
    msum = jnp.kron(eye4, mblk)
    nb2cat = jnp.tile(neb2.reshape(1, C), (1, 4))

    gu_spec = pl.BlockSpec((T4, C), lambda i: (i, 0))
    stats_spec = pl.BlockSpec((8, C), lambda i: (0, 0))
    stats_shape = jax.ShapeDtypeStruct((8, C), f32)

    p_mat, su = _pcall(
        _c1_body,
        grid=(GRID,),
        in_specs=[gu_spec, _full((C, C))],
        out_specs=[pl.BlockSpec((C, C), lambda i: (0, 0)), stats_spec],
        out_shape=[jax.ShapeDtypeStruct((C, C), f32), stats_shape],
    )(gu, msum)

    eu = _fold(su[0])[:4] / MF
    p4 = jnp.einsum('aiaj->ij', p_mat.reshape(4, QK, 4, QK))[:4, :4] / MF
    cov = p4 - jnp.outer(eu, eu)
    mu1 = geW1 @ eu + geb1
    var1 = jnp.einsum('oi,ij,oj->o', geW1, cov, geW1)
    a1 = geg1 * lax.rsqrt(var1 + 1e-5)
    c1 = gebe1 - mu1 * a1

    emb, s2 = _pcall(
        _c2_body,
        grid=(GRID,),
        in_specs=[gu_spec, gu_spec, _full((C, C)),
                  pl.BlockSpec((TILE, QK), lambda i: (i, 0)),
                  _full((C, C)), _full((1, C)), _full((1, C)), _full((1, C)),
                  _full((C, C)), _full((1, C))],
        out_specs=[gu_spec, stats_spec],
        out_shape=[jax.ShapeDtypeStruct((FP, C), f32), stats_shape],
    )(gkp, gu, msum, xq, w1blk, _tile4(geb1), _tile4(a1), _tile4(c1), w2blk,
      _tile4(geb2))

    mu2 = _fold(s2[0]) / MF
    var2 = _fold(s2[1]) / MF - mu2 * mu2
    a2 = neg0 * lax.rsqrt(var2 + 1e-5)
    c2 = nebe0 - mu2 * a2

    s3 = _pcall(
        _c3_body,
        grid=(GRID,),
        in_specs=[gu_spec, _full((1, C)), _full((1, C)), _full((C, C)),
                  _full((1, C))],
        out_specs=stats_spec,
        out_shape=stats_shape,
    )(emb, _tile4(a2), _tile4(c2), nw1blk, _tile4(neb1))

    mu3 = _fold(s3[0]) / MF
    var3 = _fold(s3[1]) / MF - mu3 * mu3
    a3 = neg1 * lax.rsqrt(var3 + 1e-5)
    c3 = nebe1 - mu3 * a3

    out = _pcall(
        _c4_body,
        grid=(GRID,),
        in_specs=([gu_spec]
                  + [pl.BlockSpec((1, T4, C), lambda i, j=j: (j, i, 0))
                     for j in range(4)]
                  + [_full((1, C)), _full((1, C)), _full((C, C)),
                     _full((1, C)), _full((1, C)), _full((1, C)),
                     _full((C, 4 * C)), _full((1, 4 * C))]),
        out_specs=pl.BlockSpec((TILE, C), lambda i: (i, 0)),
        out_shape=jax.ShapeDtypeStruct((F, C), f32),
    )(emb, gv4, gv4, gv4, gv4, _tile4(a2), _tile4(c2), nw1blk, _tile4(neb1),
      _tile4(a3), _tile4(c3), w2cat, nb2cat)

    return out.reshape(B, N, C).transpose(0, 2, 1)


# speedup vs baseline: 23.5937x; 1.0540x over previous
import functools

import jax
import jax.numpy as jnp
from jax import lax
from jax.experimental import pallas as pl
from jax.experimental.pallas import tpu as pltpu
from jax.experimental.pallas import tpu_sc as plsc

B, C, N, S = 2, 128, 10000, 16
QK = 32
NPAD = 10240
F = B * N
FS = F * S
FP = FS // 4
TILE = 400
T4 = TILE * 4
GRID = F // TILE
TN = 1024
MF = float(FS)
KXW = QK + 16
CHUNK = 128
NCHUNK = FS // CHUNK
NW = 32
ITERS = (NCHUNK + NW - 1) // NW

_dims = (((1,), (1,)), ((), ()))


def _pcall(body, **kw):
    return pl.pallas_call(body, **kw)


def _dot(a, b):
    return lax.dot_general(a, b, (((1,), (0,)), ((), ())),
                           preferred_element_type=jnp.float32)


def _proj_body(x_ref, y_ref, xyz_ref, wq_ref, bq_ref, wk_ref, bk_ref,
               wv_ref, bv_ref, xq_ref, tkx_ref, tv_ref):
    xb = x_ref[0]
    yb = y_ref[0]
    d0 = (((0,), (1,)), ((), ()))
    xq_ref[...] = lax.dot_general(xb, wq_ref[...], d0,
                                  preferred_element_type=jnp.float32) + bq_ref[...]
    k = lax.dot_general(yb, wk_ref[...], d0,
                        preferred_element_type=jnp.float32) + bk_ref[...]
    tkx_ref[...] = jnp.concatenate([k, xyz_ref[...]], axis=1)
    tv_ref[...] = lax.dot_general(yb, wv_ref[...], d0,
                                  preferred_element_type=jnp.float32) + bv_ref[...]


def _projections(x_p, y_p, xyzp, Wq, bq, Wk, bk, Wv, bv):
    nt = NPAD // TN
    return _pcall(
        _proj_body,
        grid=(B, nt),
        in_specs=[
            pl.BlockSpec((1, C, TN), lambda b, i: (b, 0, i)),
            pl.BlockSpec((1, C, TN), lambda b, i: (b, 0, i)),
            pl.BlockSpec((TN, 16), lambda b, i: (b * (NPAD // TN) + i, 0)),
            pl.BlockSpec((QK, C), lambda b, i: (0, 0)),
            pl.BlockSpec((1, QK), lambda b, i: (0, 0)),
            pl.BlockSpec((QK, C), lambda b, i: (0, 0)),
            pl.BlockSpec((1, QK), lambda b, i: (0, 0)),
            pl.BlockSpec((C, C), lambda b, i: (0, 0)),
            pl.BlockSpec((1, C), lambda b, i: (0, 0)),
        ],
        out_specs=[
            pl.BlockSpec((TN, QK), lambda b, i: (b * (NPAD // TN) + i, 0)),
            pl.BlockSpec((TN, KXW), lambda b, i: (b * (NPAD // TN) + i, 0)),
            pl.BlockSpec((TN, C), lambda b, i: (b * (NPAD // TN) + i, 0)),
        ],
        out_shape=[
            jax.ShapeDtypeStruct((B * NPAD, QK), jnp.float32),
            jax.ShapeDtypeStruct((B * NPAD, KXW), jnp.float32),
            jax.ShapeDtypeStruct((B * NPAD, C), jnp.float32),
        ],
    )(x_p, y_p, xyzp, Wq, bq, Wk, bk, Wv, bv)


def _sc_gather_kx(gidx, tkx):
    mesh = plsc.VectorSubcoreMesh(core_axis_name="c", subcore_axis_name="s")

    @functools.partial(
        pl.kernel,
        mesh=mesh,
        out_type=[
            jax.ShapeDtypeStruct((FP, C), jnp.float32),
            jax.ShapeDtypeStruct((FP, C), jnp.float32),
        ],
        scratch_types=[
            pltpu.VMEM((2, CHUNK), jnp.int32),
            pltpu.VMEM((2, CHUNK, KXW), jnp.float32),
            pltpu.VMEM((2, 32, C), jnp.float32),
            pltpu.VMEM((2, 32, C), jnp.float32),
            pltpu.VMEM((2, 8, KXW), jnp.float32),
            pltpu.SemaphoreType.DMA,
            pltpu.SemaphoreType.DMA,
        ],
        compiler_params=pltpu.CompilerParams(use_tc_tiling_on_sc=False),
    )
    def k(gidx_ref, tkx_ref, gkp_ref, gu_ref,
          idx_v, kxbuf, kpk, ubuf, cbuf, gsem, ssem):
        wid = lax.axis_index("s") * 2 + lax.axis_index("c")
        zero16 = jnp.zeros((16,), jnp.float32)

        def zbody(r, carry):
            for bb in range(2):
                for lg in range(4):
                    ubuf[bb, r, pl.ds(lg * 32 + 16, 16)] = zero16
            return carry

        lax.fori_loop(0, 32, zbody, 0)

        def fetch(cid, bb):
            base = cid * CHUNK
            pltpu.sync_copy(gidx_ref.at[pl.ds(base, CHUNK)], idx_v.at[bb])
            pltpu.async_copy(tkx_ref.at[idx_v.at[bb]], kxbuf.at[bb], gsem)
            f0 = cid * 8
            b = f0 // N
            pt0 = b * NPAD + (f0 - b * N)
            pltpu.sync_copy(tkx_ref.at[pl.ds(pt0, 8)], cbuf.at[bb])

        def drain_gather(bb):
            pltpu.make_async_copy(tkx_ref.at[pl.ds(0, CHUNK)],
                                  kxbuf.at[bb], gsem).wait()

        def drain_scatter(bb):
            pltpu.make_async_copy(kpk.at[bb], gkp_ref.at[pl.ds(0, 32)],
                                  ssem).wait()
            pltpu.make_async_copy(ubuf.at[bb], gu_ref.at[pl.ds(0, 32)],
                                  ssem).wait()

        fetch(wid, 0)

        def body(tt, carry):
            for sb in range(2):
                cid = wid + (tt * 2 + sb) * NW

                @pl.when(cid < NCHUNK)
                def _():
                    drain_gather(sb)

                    @pl.when(tt >= 1)
                    def _():
                        drain_scatter(sb)

                    def pbody(p, carry2):
                        cvec = cbuf[sb, p, pl.ds(QK, 16)]
                        for s in range(16):
                            u = p * 16 + s
                            g = kxbuf[sb, u, pl.ds(QK, 16)]
                            dv = g - cvec
                            r_out = p * 4 + s // 4
                            lg = s % 4
                            ubuf[sb, r_out, pl.ds(lg * 32, 16)] = dv
                            kpk[sb, r_out, pl.ds(lg * 32, 16)] = \
                                kxbuf[sb, u, pl.ds(0, 16)]
                            kpk[sb, r_out, pl.ds(lg * 32 + 16, 16)] = \
                                kxbuf[sb, u, pl.ds(16, 16)]
                        return carry2

                    lax.fori_loop(0, 8, pbody, 0)
                    pltpu.async_copy(kpk.at[sb],
                                     gkp_ref.at[pl.ds(cid * 32, 32)], ssem)
                    pltpu.async_copy(ubuf.at[sb],
                                     gu_ref.at[pl.ds(cid * 32, 32)], ssem)
                    ncid = cid + NW

                    @pl.when(ncid < NCHUNK)
                    def _():
                        fetch(ncid, 1 - sb)

            return carry

        lax.fori_loop(0, (ITERS + 1) // 2, body, 0)
        drain_scatter(0)
        drain_scatter(1)

    return k(gidx, tkx)


def _sc_gather_v(gidxp, tv):
    mesh = plsc.VectorSubcoreMesh(core_axis_name="c", subcore_axis_name="s")

    @functools.partial(
        pl.kernel,
        mesh=mesh,
        out_type=jax.ShapeDtypeStruct((FS, C), jnp.float32),
        scratch_types=[
            pltpu.VMEM((2, CHUNK), jnp.int32),
            pltpu.VMEM((2, CHUNK, C), jnp.float32),
            pltpu.SemaphoreType.DMA,
            pltpu.SemaphoreType.DMA,
        ],
    )
    def k(gidxp_ref, tv_ref, gv_ref, idxp_v, vbuf, gsem, ssem):
        wid = lax.axis_index("s") * 2 + lax.axis_index("c")

        def fetch(cid, bb):
            base = cid * CHUNK
            pltpu.sync_copy(gidxp_ref.at[pl.ds(base, CHUNK)], idxp_v.at[bb])
            pltpu.async_copy(tv_ref.at[idxp_v.at[bb]], vbuf.at[bb], gsem)

        def drain_scatter(bb):
            pltpu.make_async_copy(vbuf.at[bb], gv_ref.at[pl.ds(0, CHUNK)],
                                  ssem).wait()

        fetch(wid, 0)

        def body(tt, carry):
            for sb in range(2):
                cid = wid + (tt * 2 + sb) * NW

                @pl.when(cid < NCHUNK)
                def _():
                    pltpu.make_async_copy(tv_ref.at[pl.ds(0, CHUNK)],
                                          vbuf.at[sb], gsem).wait()
                    pltpu.async_copy(vbuf.at[sb],
                                     gv_ref.at[pl.ds(cid * CHUNK, CHUNK)],
                                     ssem)
                    ncid = cid + NW

                    @pl.when(ncid < NCHUNK)
                    def _():
                        if sb == 1:
                            drain_scatter(0)
                        else:
                            @pl.when(tt >= 1)
                            def _():
                                drain_scatter(1)

                        fetch(ncid, 1 - sb)

            return carry

        lax.fori_loop(0, (ITERS + 1) // 2, body, 0)
        drain_scatter(0)
        drain_scatter(1)

    return k(gidxp, tv)


def _euc(u, msum):
    sqv = _dot(u * u, msum)
    lane = lax.broadcasted_iota(jnp.int32, u.shape, 1) % QK
    return jnp.where(lane == 3, jnp.sqrt(sqv + 1e-12), u)


def _acc_stats(s_ref, v):
    @pl.when(pl.program_id(0) == 0)
    def _():
        s_ref[...] = jnp.zeros_like(s_ref)

    s_ref[0:1, :] += jnp.sum(v, axis=0)[None]
    s_ref[1:2, :] += jnp.sum(v * v, axis=0)[None]


def _c1_body(gu_ref, msum_ref, p_ref, su_ref):
    ue = _euc(gu_ref[...], msum_ref[...])
    i = pl.program_id(0)

    @pl.when(i == 0)
    def _():
        p_ref[...] = jnp.zeros_like(p_ref)
        su_ref[...] = jnp.zeros_like(su_ref)

    p_ref[...] += lax.dot_general(ue, ue, (((0,), (0,)), ((), ())),
                                  preferred_element_type=jnp.float32)
    su_ref[0:1, :] += jnp.sum(ue, axis=0)[None]


def _c2_body(gkp_ref, gu_ref, msum_ref, xq_ref, w1_ref, b1_ref, a1_ref,
             c1_ref, w2_ref, b2_ref, emb_ref, s2_ref):
    ue = _euc(gu_ref[...], msum_ref[...])
    hp = _dot(ue, w1_ref[...]) + b1_ref[...]
    h1 = jnp.maximum(hp * a1_ref[...] + c1_ref[...], 0.0)
    h = _dot(h1, w2_ref[...]) + b2_ref[...]
    q = jnp.concatenate([xq_ref[...]] * 4, axis=1)
    q4 = jnp.broadcast_to(q[:, None, :], (TILE, 4, C)).reshape(T4, C)
    d = gkp_ref[...] - q4
    emb = h - d * d
    emb_ref[...] = emb
    _acc_stats(s2_ref, emb)


def _c3_body(emb_ref, a2_ref, c2_ref, w1_ref, b1_ref, s3_ref):
    r = jnp.maximum(emb_ref[...] * a2_ref[...] + c2_ref[...], 0.0)
    t1 = _dot(r, w1_ref[...]) + b1_ref[...]
    _acc_stats(s3_ref, t1)


def _c4_body(emb_ref, gv0_ref, gv1_ref, gv2_ref, gv3_ref, a2_ref, c2_ref,
             w1_ref, b1_ref, a3_ref, c3_ref, w2_ref, b2_ref, out_ref):
    r = jnp.maximum(emb_ref[...] * a2_ref[...] + c2_ref[...], 0.0)
    t1 = _dot(r, w1_ref[...]) + b1_ref[...]
    t1 = jnp.maximum(t1 * a3_ref[...] + c3_ref[...], 0.0)
    t2 = _dot(t1, w2_ref[...]) + b2_ref[...]
    ts = [t2[:, j * C:(j + 1) * C] for j in range(4)]
    m = jnp.maximum(jnp.maximum(ts[0], ts[1]), jnp.maximum(ts[2], ts[3]))
    m = jnp.max(m.reshape(TILE, 4, C), axis=1)
    mr = jnp.broadcast_to(m[:, None, :], (TILE, 4, C)).reshape(T4, C)
    es = [jnp.exp(t - mr) for t in ts]
    den = jnp.sum((es[0] + es[1] + es[2] + es[3]).reshape(TILE, 4, C), axis=1)
    acc = (es[0] * gv0_ref[0] + es[1] * gv1_ref[0]
           + es[2] * gv2_ref[0] + es[3] * gv3_ref[0])
    out_ref[...] = jnp.sum(acc.reshape(TILE, 4, C), axis=1) / den


def _full(shape):
    return pl.BlockSpec(shape, lambda i: tuple(0 for _ in shape))


def _fold(row):
    return row.reshape(4, QK).sum(axis=0)


def _tile4(v):
    return jnp.tile(v.reshape(1, QK), (1, 4))


def kernel(x, y, y_xyz, idx, Wq, bq, Wk, bk, Wv, bv, geW1, geb1, geg1, gebe1,
           geW2, geb2, neg0, nebe0, neW1, neb1, neg1, nebe1, neW2, neb2):
    f32 = jnp.float32
    x_p = jnp.pad(x, ((0, 0), (0, 0), (0, NPAD - N)))
    y_p = jnp.pad(y, ((0, 0), (0, 0), (0, NPAD - N)))

    xyzt = jnp.transpose(y_xyz, (0, 2, 1))
    txp = jnp.pad(xyzt, ((0, 0), (0, NPAD - N), (0, 13))).reshape(B * NPAD, 16)

    xqp, tkx, tv = _projections(
        x_p, y_p, txp, Wq, bq.reshape(1, QK).astype(f32), Wk,
        bk.reshape(1, QK).astype(f32), Wv, bv.reshape(1, C).astype(f32))
    xq = xqp.reshape(B, NPAD, QK)[:, :N].reshape(F, QK)

    gidx = (idx.astype(jnp.int32)
            + (jnp.arange(B, dtype=jnp.int32) * NPAD)[:, None, None]
            ).reshape(FS)
    gidxp = gidx.reshape(FP, 4).T.reshape(FS)

    gkp, gu = _sc_gather_kx(gidx, tkx)
    gv = _sc_gather_v(gidxp, tv)
    gv4 = gv.reshape(4, FP, C)

    eye4 = jnp.eye(4, dtype=f32)
    w1blk = jnp.kron(eye4, jnp.pad(geW1.T, ((0, QK - 4), (0, 0))))
    w2blk = jnp.kron(eye4, geW2.T)
    nw1blk = jnp.kron(eye4, neW1.T)
    w2cat = jnp.concatenate(
        [jnp.pad(neW2.T, ((QK * j, QK * (3 - j)), (0, 0))) for j in range(4)],
        axis=1)
    mblk = jnp.zeros((QK, QK), f32).at[0:3, 3].set(1.0)
    msum = jnp.kron(eye4, mblk)
    nb2cat = jnp.tile(neb2.reshape(1, C), (1, 4))

    gu_spec = pl.BlockSpec((T4, C), lambda i: (i, 0))
    stats_spec = pl.BlockSpec((8, C), lambda i: (0, 0))
    stats_shape = jax.ShapeDtypeStruct((8, C), f32)

    p_mat, su = _pcall(
        _c1_body,
        grid=(GRID,),
        in_specs=[gu_spec, _full((C, C))],
        out_specs=[pl.BlockSpec((C, C), lambda i: (0, 0)), stats_spec],
        out_shape=[jax.ShapeDtypeStruct((C, C), f32), stats_shape],
    )(gu, msum)

    eu = _fold(su[0])[:4] / MF
    p4 = jnp.einsum('aiaj->ij', p_mat.reshape(4, QK, 4, QK))[:4, :4] / MF
    cov = p4 - jnp.outer(eu, eu)
    mu1 = geW1 @ eu + geb1
    var1 = jnp.einsum('oi,ij,oj->o', geW1, cov, geW1)
    a1 = geg1 * lax.rsqrt(var1 + 1e-5)
    c1 = gebe1 - mu1 * a1

    emb, s2 = _pcall(
        _c2_body,
        grid=(GRID,),
        in_specs=[gu_spec, gu_spec, _full((C, C)),
                  pl.BlockSpec((TILE, QK), lambda i: (i, 0)),
                  _full((C, C)), _full((1, C)), _full((1, C)), _full((1, C)),
                  _full((C, C)), _full((1, C))],
        out_specs=[gu_spec, stats_spec],
        out_shape=[jax.ShapeDtypeStruct((FP, C), f32), stats_shape],
    )(gkp, gu, msum, xq, w1blk, _tile4(geb1), _tile4(a1), _tile4(c1), w2blk,
      _tile4(geb2))

    mu2 = _fold(s2[0]) / MF
    var2 = _fold(s2[1]) / MF - mu2 * mu2
    a2 = neg0 * lax.rsqrt(var2 + 1e-5)
    c2 = nebe0 - mu2 * a2

    s3 = _pcall(
        _c3_body,
        grid=(GRID,),
        in_specs=[gu_spec, _full((1, C)), _full((1, C)), _full((C, C)),
                  _full((1, C))],
        out_specs=stats_spec,
        out_shape=stats_shape,
    )(emb, _tile4(a2), _tile4(c2), nw1blk, _tile4(neb1))

    mu3 = _fold(s3[0]) / MF
    var3 = _fold(s3[1]) / MF - mu3 * mu3
    a3 = neg1 * lax.rsqrt(var3 + 1e-5)
    c3 = nebe1 - mu3 * a3

    out = _pcall(
        _c4_body,
        grid=(GRID,),
        in_specs=([gu_spec]
                  + [pl.BlockSpec((1, T4, C), lambda i, j=j: (j, i, 0))
                     for j in range(4)]
                  + [_full((1, C)), _full((1, C)), _full((C, C)),
                     _full((1, C)), _full((1, C)), _full((1, C)),
                     _full((C, 4 * C)), _full((1, 4 * C))]),
        out_specs=pl.BlockSpec((TILE, C), lambda i: (i, 0)),
        out_shape=jax.ShapeDtypeStruct((F, C), f32),
    )(emb, gv4, gv4, gv4, gv4, _tile4(a2), _tile4(c2), nw1blk, _tile4(neb1),
      _tile4(a3), _tile4(c3), w2cat, nb2cat)

    return out.reshape(B, N, C).transpose(0, 2, 1)


# speedup vs baseline: 26.7442x; 1.1335x over previous
import functools

import jax
import jax.numpy as jnp
from jax import lax
from jax.experimental import pallas as pl
from jax.experimental.pallas import tpu as pltpu
from jax.experimental.pallas import tpu_sc as plsc

B, C, N, S = 2, 128, 10000, 16
QK = 32
NPAD = 10240
F = B * N
FS = F * S
FP = FS // 4
TILE = 400
T4 = TILE * 4
GRID = F // TILE
TN = 1024
MF = float(FS)
KXW = QK + 16
CHUNK = 128
NCHUNK = FS // CHUNK
NW = 32
ITERS = (NCHUNK + NW - 1) // NW

_dims = (((1,), (1,)), ((), ()))


def _pcall(body, **kw):
    return pl.pallas_call(body, **kw)


def _dot(a, b):
    return lax.dot_general(a, b, (((1,), (0,)), ((), ())),
                           preferred_element_type=jnp.float32)


def _proj_body(x_ref, y_ref, xyz_ref, wq_ref, bq_ref, wk_ref, bk_ref,
               wv_ref, bv_ref, xq_ref, tkx_ref, tv_ref):
    xb = x_ref[0]
    yb = y_ref[0]
    d0 = (((0,), (1,)), ((), ()))
    xq_ref[...] = lax.dot_general(xb, wq_ref[...], d0,
                                  preferred_element_type=jnp.float32) + bq_ref[...]
    k = lax.dot_general(yb, wk_ref[...], d0,
                        preferred_element_type=jnp.float32) + bk_ref[...]
    tkx_ref[...] = jnp.concatenate([k, xyz_ref[...]], axis=1)
    tv_ref[...] = lax.dot_general(yb, wv_ref[...], d0,
                                  preferred_element_type=jnp.float32) + bv_ref[...]


def _projections(x_p, y_p, xyzp, Wq, bq, Wk, bk, Wv, bv):
    nt = NPAD // TN
    return _pcall(
        _proj_body,
        grid=(B, nt),
        in_specs=[
            pl.BlockSpec((1, C, TN), lambda b, i: (b, 0, i)),
            pl.BlockSpec((1, C, TN), lambda b, i: (b, 0, i)),
            pl.BlockSpec((TN, 16), lambda b, i: (b * (NPAD // TN) + i, 0)),
            pl.BlockSpec((QK, C), lambda b, i: (0, 0)),
            pl.BlockSpec((1, QK), lambda b, i: (0, 0)),
            pl.BlockSpec((QK, C), lambda b, i: (0, 0)),
            pl.BlockSpec((1, QK), lambda b, i: (0, 0)),
            pl.BlockSpec((C, C), lambda b, i: (0, 0)),
            pl.BlockSpec((1, C), lambda b, i: (0, 0)),
        ],
        out_specs=[
            pl.BlockSpec((TN, QK), lambda b, i: (b * (NPAD // TN) + i, 0)),
            pl.BlockSpec((TN, KXW), lambda b, i: (b * (NPAD // TN) + i, 0)),
            pl.BlockSpec((TN, C), lambda b, i: (b * (NPAD // TN) + i, 0)),
        ],
        out_shape=[
            jax.ShapeDtypeStruct((B * NPAD, QK), jnp.float32),
            jax.ShapeDtypeStruct((B * NPAD, KXW), jnp.float32),
            jax.ShapeDtypeStruct((B * NPAD, C), jnp.float32),
        ],
    )(x_p, y_p, xyzp, Wq, bq, Wk, bk, Wv, bv)


def _sc_gather_kx(gidx, tkx):
    mesh = plsc.VectorSubcoreMesh(core_axis_name="c", subcore_axis_name="s")

    @functools.partial(
        pl.kernel,
        mesh=mesh,
        out_type=[
            jax.ShapeDtypeStruct((FP, C), jnp.float32),
            jax.ShapeDtypeStruct((FP, C), jnp.float32),
        ],
        scratch_types=[
            pltpu.VMEM((2, CHUNK), jnp.int32),
            pltpu.VMEM((2, CHUNK, KXW), jnp.float32),
            pltpu.VMEM((2, 32, C), jnp.float32),
            pltpu.VMEM((2, 32, C), jnp.float32),
            pltpu.VMEM((2, 8, KXW), jnp.float32),
            pltpu.SemaphoreType.DMA,
            pltpu.SemaphoreType.DMA,
        ],
        compiler_params=pltpu.CompilerParams(use_tc_tiling_on_sc=False),
    )
    def k(gidx_ref, tkx_ref, gkp_ref, gu_ref,
          idx_v, kxbuf, kpk, ubuf, cbuf, gsem, ssem):
        wid = lax.axis_index("s") * 2 + lax.axis_index("c")
        zero16 = jnp.zeros((16,), jnp.float32)

        def zbody(r, carry):
            for bb in range(2):
                for lg in range(4):
                    ubuf[bb, r, pl.ds(lg * 32 + 16, 16)] = zero16
            return carry

        lax.fori_loop(0, 32, zbody, 0)

        def fetch(cid, bb):
            base = cid * CHUNK
            pltpu.sync_copy(gidx_ref.at[pl.ds(base, CHUNK)], idx_v.at[bb])
            pltpu.async_copy(tkx_ref.at[idx_v.at[bb]], kxbuf.at[bb], gsem)
            f0 = cid * 8
            b = f0 // N
            pt0 = b * NPAD + (f0 - b * N)
            pltpu.sync_copy(tkx_ref.at[pl.ds(pt0, 8)], cbuf.at[bb])

        def drain_gather(bb):
            pltpu.make_async_copy(tkx_ref.at[pl.ds(0, CHUNK)],
                                  kxbuf.at[bb], gsem).wait()

        def drain_scatter(bb):
            pltpu.make_async_copy(kpk.at[bb], gkp_ref.at[pl.ds(0, 32)],
                                  ssem).wait()
            pltpu.make_async_copy(ubuf.at[bb], gu_ref.at[pl.ds(0, 32)],
                                  ssem).wait()

        fetch(wid, 0)

        def body(tt, carry):
            for sb in range(2):
                cid = wid + (tt * 2 + sb) * NW

                @pl.when(cid < NCHUNK)
                def _():
                    drain_gather(sb)

                    @pl.when(tt >= 1)
                    def _():
                        drain_scatter(sb)

                    for p in range(8):
                        cvec = cbuf[sb, p, pl.ds(QK, 16)]
                        for s in range(16):
                            u = p * 16 + s
                            g = kxbuf[sb, u, pl.ds(QK, 16)]
                            dv = g - cvec
                            r_out = p * 4 + s // 4
                            lg = s % 4
                            ubuf[sb, r_out, pl.ds(lg * 32, 16)] = dv
                            kpk[sb, r_out, pl.ds(lg * 32, 16)] = \
                                kxbuf[sb, u, pl.ds(0, 16)]
                            kpk[sb, r_out, pl.ds(lg * 32 + 16, 16)] = \
                                kxbuf[sb, u, pl.ds(16, 16)]
                    pltpu.async_copy(kpk.at[sb],
                                     gkp_ref.at[pl.ds(cid * 32, 32)], ssem)
                    pltpu.async_copy(ubuf.at[sb],
                                     gu_ref.at[pl.ds(cid * 32, 32)], ssem)
                    ncid = cid + NW

                    @pl.when(ncid < NCHUNK)
                    def _():
                        fetch(ncid, 1 - sb)

            return carry

        lax.fori_loop(0, (ITERS + 1) // 2, body, 0)
        drain_scatter(0)
        drain_scatter(1)

    return k(gidx, tkx)


def _sc_gather_v(gidxp, tv):
    mesh = plsc.VectorSubcoreMesh(core_axis_name="c", subcore_axis_name="s")

    @functools.partial(
        pl.kernel,
        mesh=mesh,
        out_type=jax.ShapeDtypeStruct((FS, C), jnp.float32),
        scratch_types=[
            pltpu.VMEM((2, CHUNK), jnp.int32),
            pltpu.VMEM((2, CHUNK, C), jnp.float32),
            pltpu.SemaphoreType.DMA,
            pltpu.SemaphoreType.DMA,
        ],
    )
    def k(gidxp_ref, tv_ref, gv_ref, idxp_v, vbuf, gsem, ssem):
        wid = lax.axis_index("s") * 2 + lax.axis_index("c")

        def fetch(cid, bb):
            base = cid * CHUNK
            pltpu.sync_copy(gidxp_ref.at[pl.ds(base, CHUNK)], idxp_v.at[bb])
            pltpu.async_copy(tv_ref.at[idxp_v.at[bb]], vbuf.at[bb], gsem)

        def drain_scatter(bb):
            pltpu.make_async_copy(vbuf.at[bb], gv_ref.at[pl.ds(0, CHUNK)],
                                  ssem).wait()

        fetch(wid, 0)

        def body(tt, carry):
            for sb in range(2):
                cid = wid + (tt * 2 + sb) * NW

                @pl.when(cid < NCHUNK)
                def _():
                    pltpu.make_async_copy(tv_ref.at[pl.ds(0, CHUNK)],
                                          vbuf.at[sb], gsem).wait()
                    pltpu.async_copy(vbuf.at[sb],
                                     gv_ref.at[pl.ds(cid * CHUNK, CHUNK)],
                                     ssem)
                    ncid = cid + NW

                    @pl.when(ncid < NCHUNK)
                    def _():
                        if sb == 1:
                            drain_scatter(0)
                        else:
                            @pl.when(tt >= 1)
                            def _():
                                drain_scatter(1)

                        fetch(ncid, 1 - sb)

            return carry

        lax.fori_loop(0, (ITERS + 1) // 2, body, 0)
        drain_scatter(0)
        drain_scatter(1)

    return k(gidxp, tv)


def _euc(u, msum):
    sqv = _dot(u * u, msum)
    lane = lax.broadcasted_iota(jnp.int32, u.shape, 1) % QK
    return jnp.where(lane == 3, jnp.sqrt(sqv + 1e-12), u)


def _acc_stats(s_ref, v):
    @pl.when(pl.program_id(0) == 0)
    def _():
        s_ref[...] = jnp.zeros_like(s_ref)

    s_ref[0:1, :] += jnp.sum(v, axis=0)[None]
    s_ref[1:2, :] += jnp.sum(v * v, axis=0)[None]


def _c1_body(gu_ref, msum_ref, p_ref, su_ref):
    ue = _euc(gu_ref[...], msum_ref[...])
    i = pl.program_id(0)

    @pl.when(i == 0)
    def _():
        p_ref[...] = jnp.zeros_like(p_ref)
        su_ref[...] = jnp.zeros_like(su_ref)

    p_ref[...] += lax.dot_general(ue, ue, (((0,), (0,)), ((), ())),
                                  preferred_element_type=jnp.float32)
    su_ref[0:1, :] += jnp.sum(ue, axis=0)[None]


def _c2_body(gkp_ref, gu_ref, msum_ref, xq_ref, w1_ref, b1_ref, a1_ref,
             c1_ref, w2_ref, b2_ref, emb_ref, s2_ref):
    ue = _euc(gu_ref[...], msum_ref[...])
    hp = _dot(ue, w1_ref[...]) + b1_ref[...]
    h1 = jnp.maximum(hp * a1_ref[...] + c1_ref[...], 0.0)
    h = _dot(h1, w2_ref[...]) + b2_ref[...]
    q = jnp.concatenate([xq_ref[...]] * 4, axis=1)
    q4 = jnp.broadcast_to(q[:, None, :], (TILE, 4, C)).reshape(T4, C)
    d = gkp_ref[...] - q4
    emb = h - d * d
    emb_ref[...] = emb
    _acc_stats(s2_ref, emb)


def _c3_body(emb_ref, a2_ref, c2_ref, w1_ref, b1_ref, s3_ref):
    r = jnp.maximum(emb_ref[...] * a2_ref[...] + c2_ref[...], 0.0)
    t1 = _dot(r, w1_ref[...]) + b1_ref[...]
    _acc_stats(s3_ref, t1)


def _c4_body(emb_ref, gv0_ref, gv1_ref, gv2_ref, gv3_ref, a2_ref, c2_ref,
             w1_ref, b1_ref, a3_ref, c3_ref, w2_ref, b2_ref, out_ref):
    r = jnp.maximum(emb_ref[...] * a2_ref[...] + c2_ref[...], 0.0)
    t1 = _dot(r, w1_ref[...]) + b1_ref[...]
    t1 = jnp.maximum(t1 * a3_ref[...] + c3_ref[...], 0.0)
    t2 = _dot(t1, w2_ref[...]) + b2_ref[...]
    ts = [t2[:, j * C:(j + 1) * C] for j in range(4)]
    m = jnp.maximum(jnp.maximum(ts[0], ts[1]), jnp.maximum(ts[2], ts[3]))
    m = jnp.max(m.reshape(TILE, 4, C), axis=1)
    mr = jnp.broadcast_to(m[:, None, :], (TILE, 4, C)).reshape(T4, C)
    es = [jnp.exp(t - mr) for t in ts]
    den = jnp.sum((es[0] + es[1] + es[2] + es[3]).reshape(TILE, 4, C), axis=1)
    acc = (es[0] * gv0_ref[0] + es[1] * gv1_ref[0]
           + es[2] * gv2_ref[0] + es[3] * gv3_ref[0])
    out_ref[...] = jnp.sum(acc.reshape(TILE, 4, C), axis=1) / den


def _full(shape):
    return pl.BlockSpec(shape, lambda i: tuple(0 for _ in shape))


def _fold(row):
    return row.reshape(4, QK).sum(axis=0)


def _tile4(v):
    return jnp.tile(v.reshape(1, QK), (1, 4))


def kernel(x, y, y_xyz, idx, Wq, bq, Wk, bk, Wv, bv, geW1, geb1, geg1, gebe1,
           geW2, geb2, neg0, nebe0, neW1, neb1, neg1, nebe1, neW2, neb2):
    f32 = jnp.float32
    x_p = jnp.pad(x, ((0, 0), (0, 0), (0, NPAD - N)))
    y_p = jnp.pad(y, ((0, 0), (0, 0), (0, NPAD - N)))

    xyzt = jnp.transpose(y_xyz, (0, 2, 1))
    txp = jnp.pad(xyzt, ((0, 0), (0, NPAD - N), (0, 13))).reshape(B * NPAD, 16)

    xqp, tkx, tv = _projections(
        x_p, y_p, txp, Wq, bq.reshape(1, QK).astype(f32), Wk,
        bk.reshape(1, QK).astype(f32), Wv, bv.reshape(1, C).astype(f32))
    xq = xqp.reshape(B, NPAD, QK)[:, :N].reshape(F, QK)

    gidx = (idx.astype(jnp.int32)
            + (jnp.arange(B, dtype=jnp.int32) * NPAD)[:, None, None]
            ).reshape(FS)
    gidxp = gidx.reshape(FP, 4).T.reshape(FS)

    gkp, gu = _sc_gather_kx(gidx, tkx)
    gv = _sc_gather_v(gidxp, tv)
    gv4 = gv.reshape(4, FP, C)

    eye4 = jnp.eye(4, dtype=f32)
    w1blk = jnp.kron(eye4, jnp.pad(geW1.T, ((0, QK - 4), (0, 0))))
    w2blk = jnp.kron(eye4, geW2.T)
    nw1blk = jnp.kron(eye4, neW1.T)
    w2cat = jnp.concatenate(
        [jnp.pad(neW2.T, ((QK * j, QK * (3 - j)), (0, 0))) for j in range(4)],
        axis=1)
    mblk = jnp.zeros((QK, QK), f32).at[0:3, 3].set(1.0)
    msum = jnp.kron(eye4, mblk)
    nb2cat = jnp.tile(neb2.reshape(1, C), (1, 4))

    gu_spec = pl.BlockSpec((T4, C), lambda i: (i, 0))
    stats_spec = pl.BlockSpec((8, C), lambda i: (0, 0))
    stats_shape = jax.ShapeDtypeStruct((8, C), f32)

    p_mat, su = _pcall(
        _c1_body,
        grid=(GRID,),
        in_specs=[gu_spec, _full((C, C))],
        out_specs=[pl.BlockSpec((C, C), lambda i: (0, 0)), stats_spec],
        out_shape=[jax.ShapeDtypeStruct((C, C), f32), stats_shape],
    )(gu, msum)

    eu = _fold(su[0])[:4] / MF
    p4 = jnp.einsum('aiaj->ij', p_mat.reshape(4, QK, 4, QK))[:4, :4] / MF
    cov = p4 - jnp.outer(eu, eu)
    mu1 = geW1 @ eu + geb1
    var1 = jnp.einsum('oi,ij,oj->o', geW1, cov, geW1)
    a1 = geg1 * lax.rsqrt(var1 + 1e-5)
    c1 = gebe1 - mu1 * a1

    emb, s2 = _pcall(
        _c2_body,
        grid=(GRID,),
        in_specs=[gu_spec, gu_spec, _full((C, C)),
                  pl.BlockSpec((TILE, QK), lambda i: (i, 0)),
                  _full((C, C)), _full((1, C)), _full((1, C)), _full((1, C)),
                  _full((C, C)), _full((1, C))],
        out_specs=[gu_spec, stats_spec],
        out_shape=[jax.ShapeDtypeStruct((FP, C), f32), stats_shape],
    )(gkp, gu, msum, xq, w1blk, _tile4(geb1), _tile4(a1), _tile4(c1), w2blk,
      _tile4(geb2))

    mu2 = _fold(s2[0]) / MF
    var2 = _fold(s2[1]) / MF - mu2 * mu2
    a2 = neg0 * lax.rsqrt(var2 + 1e-5)
    c2 = nebe0 - mu2 * a2

    s3 = _pcall(
        _c3_body,
        grid=(GRID,),
        in_specs=[gu_spec, _full((1, C)), _full((1, C)), _full((C, C)),
                  _full((1, C))],
        out_specs=stats_spec,
        out_shape=stats_shape,
    )(emb, _tile4(a2), _tile4(c2), nw1blk, _tile4(neb1))

    mu3 = _fold(s3[0]) / MF
    var3 = _fold(s3[1]) / MF - mu3 * mu3
    a3 = neg1 * lax.rsqrt(var3 + 1e-5)
    c3 = nebe1 - mu3 * a3

    out = _pcall(
        _c4_body,
        grid=(GRID,),
        in_specs=([gu_spec]
                  + [pl.BlockSpec((1, T4, C), lambda i, j=j: (j, i, 0))
                     for j in range(4)]
                  + [_full((1, C)), _full((1, C)), _full((C, C)),
                     _full((1, C)), _full((1, C)), _full((1, C)),
                     _full((C, 4 * C)), _full((1, 4 * C))]),
        out_specs=pl.BlockSpec((TILE, C), lambda i: (i, 0)),
        out_shape=jax.ShapeDtypeStruct((F, C), f32),
    )(emb, gv4, gv4, gv4, gv4, _tile4(a2), _tile4(c2), nw1blk, _tile4(neb1),
      _tile4(a3), _tile4(c3), w2cat, nb2cat)

    return out.reshape(B, N, C).transpose(0, 2, 1)


# speedup vs baseline: 27.5383x; 1.0297x over previous
import functools

import jax
import jax.numpy as jnp
from jax import lax
from jax.experimental import pallas as pl
from jax.experimental.pallas import tpu as pltpu
from jax.experimental.pallas import tpu_sc as plsc

B, C, N, S = 2, 128, 10000, 16
QK = 32
NPAD = 10240
F = B * N
FS = F * S
FP = FS // 4
TILE = 800
T4 = TILE * 4
GRID = F // TILE
TN = 1024
MF = float(FS)
KXW = QK + 16
CHUNK = 128
NCHUNK = FS // CHUNK
NW = 32
ITERS = (NCHUNK + NW - 1) // NW

_dims = (((1,), (1,)), ((), ()))


def _pcall(body, **kw):
    return pl.pallas_call(body, **kw)


def _dot(a, b):
    return lax.dot_general(a, b, (((1,), (0,)), ((), ())),
                           preferred_element_type=jnp.float32)


def _proj_body(x_ref, y_ref, xyz_ref, wq_ref, bq_ref, wk_ref, bk_ref,
               wv_ref, bv_ref, xq_ref, tkx_ref, tv_ref):
    xb = x_ref[0]
    yb = y_ref[0]
    d0 = (((0,), (1,)), ((), ()))
    xq_ref[...] = lax.dot_general(xb, wq_ref[...], d0,
                                  preferred_element_type=jnp.float32) + bq_ref[...]
    k = lax.dot_general(yb, wk_ref[...], d0,
                        preferred_element_type=jnp.float32) + bk_ref[...]
    tkx_ref[...] = jnp.concatenate([k, xyz_ref[...]], axis=1)
    tv_ref[...] = lax.dot_general(yb, wv_ref[...], d0,
                                  preferred_element_type=jnp.float32) + bv_ref[...]


def _projections(x_p, y_p, xyzp, Wq, bq, Wk, bk, Wv, bv):
    nt = NPAD // TN
    return _pcall(
        _proj_body,
        grid=(B, nt),
        in_specs=[
            pl.BlockSpec((1, C, TN), lambda b, i: (b, 0, i)),
            pl.BlockSpec((1, C, TN), lambda b, i: (b, 0, i)),
            pl.BlockSpec((TN, 16), lambda b, i: (b * (NPAD // TN) + i, 0)),
            pl.BlockSpec((QK, C), lambda b, i: (0, 0)),
            pl.BlockSpec((1, QK), lambda b, i: (0, 0)),
            pl.BlockSpec((QK, C), lambda b, i: (0, 0)),
            pl.BlockSpec((1, QK), lambda b, i: (0, 0)),
            pl.BlockSpec((C, C), lambda b, i: (0, 0)),
            pl.BlockSpec((1, C), lambda b, i: (0, 0)),
        ],
        out_specs=[
            pl.BlockSpec((TN, QK), lambda b, i: (b * (NPAD // TN) + i, 0)),
            pl.BlockSpec((TN, KXW), lambda b, i: (b * (NPAD // TN) + i, 0)),
            pl.BlockSpec((TN, C), lambda b, i: (b * (NPAD // TN) + i, 0)),
        ],
        out_shape=[
            jax.ShapeDtypeStruct((B * NPAD, QK), jnp.float32),
            jax.ShapeDtypeStruct((B * NPAD, KXW), jnp.float32),
            jax.ShapeDtypeStruct((B * NPAD, C), jnp.float32),
        ],
    )(x_p, y_p, xyzp, Wq, bq, Wk, bk, Wv, bv)


def _sc_gather_kx(gidx, tkx):
    mesh = plsc.VectorSubcoreMesh(core_axis_name="c", subcore_axis_name="s")

    @functools.partial(
        pl.kernel,
        mesh=mesh,
        out_type=[
            jax.ShapeDtypeStruct((FP, C), jnp.float32),
            jax.ShapeDtypeStruct((FP, C), jnp.float32),
        ],
        scratch_types=[
            pltpu.VMEM((2, CHUNK), jnp.int32),
            pltpu.VMEM((2, CHUNK, KXW), jnp.float32),
            pltpu.VMEM((2, 32, C), jnp.float32),
            pltpu.VMEM((2, 32, C), jnp.float32),
            pltpu.VMEM((2, 8, KXW), jnp.float32),
            pltpu.SemaphoreType.DMA,
            pltpu.SemaphoreType.DMA,
        ],
        compiler_params=pltpu.CompilerParams(use_tc_tiling_on_sc=False),
    )
    def k(gidx_ref, tkx_ref, gkp_ref, gu_ref,
          idx_v, kxbuf, kpk, ubuf, cbuf, gsem, ssem):
        wid = lax.axis_index("s") * 2 + lax.axis_index("c")
        zero16 = jnp.zeros((16,), jnp.float32)

        def zbody(r, carry):
            for bb in range(2):
                for lg in range(4):
                    ubuf[bb, r, pl.ds(lg * 32 + 16, 16)] = zero16
            return carry

        lax.fori_loop(0, 32, zbody, 0)

        def fetch(cid, bb):
            base = cid * CHUNK
            pltpu.sync_copy(gidx_ref.at[pl.ds(base, CHUNK)], idx_v.at[bb])
            pltpu.async_copy(tkx_ref.at[idx_v.at[bb]], kxbuf.at[bb], gsem)
            f0 = cid * 8
            b = f0 // N
            pt0 = b * NPAD + (f0 - b * N)
            pltpu.sync_copy(tkx_ref.at[pl.ds(pt0, 8)], cbuf.at[bb])

        def drain_gather(bb):
            pltpu.make_async_copy(tkx_ref.at[pl.ds(0, CHUNK)],
                                  kxbuf.at[bb], gsem).wait()

        def drain_scatter(bb):
            pltpu.make_async_copy(kpk.at[bb], gkp_ref.at[pl.ds(0, 32)],
                                  ssem).wait()
            pltpu.make_async_copy(ubuf.at[bb], gu_ref.at[pl.ds(0, 32)],
                                  ssem).wait()

        fetch(wid, 0)

        def body(tt, carry):
            for sb in range(2):
                cid = wid + (tt * 2 + sb) * NW

                @pl.when(cid < NCHUNK)
                def _():
                    drain_gather(sb)

                    @pl.when(tt >= 1)
                    def _():
                        drain_scatter(sb)

                    for p in range(8):
                        cvec = cbuf[sb, p, pl.ds(QK, 16)]
                        for s in range(16):
                            u = p * 16 + s
                            g = kxbuf[sb, u, pl.ds(QK, 16)]
                            dv = g - cvec
                            r_out = p * 4 + s // 4
                            lg = s % 4
                            ubuf[sb, r_out, pl.ds(lg * 32, 16)] = dv
                            kpk[sb, r_out, pl.ds(lg * 32, 16)] = \
                                kxbuf[sb, u, pl.ds(0, 16)]
                            kpk[sb, r_out, pl.ds(lg * 32 + 16, 16)] = \
                                kxbuf[sb, u, pl.ds(16, 16)]
                    pltpu.async_copy(kpk.at[sb],
                                     gkp_ref.at[pl.ds(cid * 32, 32)], ssem)
                    pltpu.async_copy(ubuf.at[sb],
                                     gu_ref.at[pl.ds(cid * 32, 32)], ssem)
                    ncid = cid + NW

                    @pl.when(ncid < NCHUNK)
                    def _():
                        fetch(ncid, 1 - sb)

            return carry

        lax.fori_loop(0, (ITERS + 1) // 2, body, 0)
        drain_scatter(0)
        drain_scatter(1)

    return k(gidx, tkx)


def _sc_gather_v(gidxp, tv):
    mesh = plsc.VectorSubcoreMesh(core_axis_name="c", subcore_axis_name="s")

    @functools.partial(
        pl.kernel,
        mesh=mesh,
        out_type=jax.ShapeDtypeStruct((FS, C), jnp.float32),
        scratch_types=[
            pltpu.VMEM((2, CHUNK), jnp.int32),
            pltpu.VMEM((2, CHUNK, C), jnp.float32),
            pltpu.SemaphoreType.DMA,
            pltpu.SemaphoreType.DMA,
        ],
    )
    def k(gidxp_ref, tv_ref, gv_ref, idxp_v, vbuf, gsem, ssem):
        wid = lax.axis_index("s") * 2 + lax.axis_index("c")

        def fetch(cid, bb):
            base = cid * CHUNK
            pltpu.sync_copy(gidxp_ref.at[pl.ds(base, CHUNK)], idxp_v.at[bb])
            pltpu.async_copy(tv_ref.at[idxp_v.at[bb]], vbuf.at[bb], gsem)

        def drain_scatter(bb):
            pltpu.make_async_copy(vbuf.at[bb], gv_ref.at[pl.ds(0, CHUNK)],
                                  ssem).wait()

        fetch(wid, 0)

        def body(tt, carry):
            for sb in range(2):
                cid = wid + (tt * 2 + sb) * NW

                @pl.when(cid < NCHUNK)
                def _():
                    pltpu.make_async_copy(tv_ref.at[pl.ds(0, CHUNK)],
                                          vbuf.at[sb], gsem).wait()
                    pltpu.async_copy(vbuf.at[sb],
                                     gv_ref.at[pl.ds(cid * CHUNK, CHUNK)],
                                     ssem)
                    ncid = cid + NW

                    @pl.when(ncid < NCHUNK)
                    def _():
                        if sb == 1:
                            drain_scatter(0)
                        else:
                            @pl.when(tt >= 1)
                            def _():
                                drain_scatter(1)

                        fetch(ncid, 1 - sb)

            return carry

        lax.fori_loop(0, (ITERS + 1) // 2, body, 0)
        drain_scatter(0)
        drain_scatter(1)

    return k(gidxp, tv)


def _euc(u, msum):
    sqv = _dot(u * u, msum)
    lane = lax.broadcasted_iota(jnp.int32, u.shape, 1) % QK
    return jnp.where(lane == 3, jnp.sqrt(sqv + 1e-12), u)


def _acc_stats(s_ref, v):
    @pl.when(pl.program_id(0) == 0)
    def _():
        s_ref[...] = jnp.zeros_like(s_ref)

    s_ref[0:1, :] += jnp.sum(v, axis=0)[None]
    s_ref[1:2, :] += jnp.sum(v * v, axis=0)[None]


def _c1_body(gu_ref, msum_ref, p_ref, su_ref):
    ue = _euc(gu_ref[...], msum_ref[...])
    i = pl.program_id(0)

    @pl.when(i == 0)
    def _():
        p_ref[...] = jnp.zeros_like(p_ref)
        su_ref[...] = jnp.zeros_like(su_ref)

    p_ref[...] += lax.dot_general(ue, ue, (((0,), (0,)), ((), ())),
                                  preferred_element_type=jnp.float32)
    su_ref[0:1, :] += jnp.sum(ue, axis=0)[None]


def _c2_body(gkp_ref, gu_ref, msum_ref, xq_ref, w1_ref, b1_ref, a1_ref,
             c1_ref, w2_ref, b2_ref, emb_ref, s2_ref):
    ue = _euc(gu_ref[...], msum_ref[...])
    hp = _dot(ue, w1_ref[...]) + b1_ref[...]
    h1 = jnp.maximum(hp * a1_ref[...] + c1_ref[...], 0.0)
    h = _dot(h1, w2_ref[...]) + b2_ref[...]
    q = jnp.concatenate([xq_ref[...]] * 4, axis=1)
    q4 = jnp.broadcast_to(q[:, None, :], (TILE, 4, C)).reshape(T4, C)
    d = gkp_ref[...] - q4
    emb = h - d * d
    emb_ref[...] = emb
    _acc_stats(s2_ref, emb)


def _c3_body(emb_ref, a2_ref, c2_ref, w1_ref, b1_ref, s3_ref):
    r = jnp.maximum(emb_ref[...] * a2_ref[...] + c2_ref[...], 0.0)
    t1 = _dot(r, w1_ref[...]) + b1_ref[...]
    _acc_stats(s3_ref, t1)


def _c4_body(emb_ref, gv0_ref, gv1_ref, gv2_ref, gv3_ref, a2_ref, c2_ref,
             w1_ref, b1_ref, a3_ref, c3_ref, w2_ref, b2_ref, out_ref):
    r = jnp.maximum(emb_ref[...] * a2_ref[...] + c2_ref[...], 0.0)
    t1 = _dot(r, w1_ref[...]) + b1_ref[...]
    t1 = jnp.maximum(t1 * a3_ref[...] + c3_ref[...], 0.0)
    t2 = _dot(t1, w2_ref[...]) + b2_ref[...]
    ts = [t2[:, j * C:(j + 1) * C] for j in range(4)]
    m = jnp.maximum(jnp.maximum(ts[0], ts[1]), jnp.maximum(ts[2], ts[3]))
    m = jnp.max(m.reshape(TILE, 4, C), axis=1)
    mr = jnp.broadcast_to(m[:, None, :], (TILE, 4, C)).reshape(T4, C)
    es = [jnp.exp(t - mr) for t in ts]
    den = jnp.sum((es[0] + es[1] + es[2] + es[3]).reshape(TILE, 4, C), axis=1)
    acc = (es[0] * gv0_ref[0] + es[1] * gv1_ref[0]
           + es[2] * gv2_ref[0] + es[3] * gv3_ref[0])
    out_ref[...] = jnp.sum(acc.reshape(TILE, 4, C), axis=1) / den


def _full(shape):
    return pl.BlockSpec(shape, lambda i: tuple(0 for _ in shape))


def _fold(row):
    return row.reshape(4, QK).sum(axis=0)


def _tile4(v):
    return jnp.tile(v.reshape(1, QK), (1, 4))


def kernel(x, y, y_xyz, idx, Wq, bq, Wk, bk, Wv, bv, geW1, geb1, geg1, gebe1,
           geW2, geb2, neg0, nebe0, neW1, neb1, neg1, nebe1, neW2, neb2):
    f32 = jnp.float32
    x_p = jnp.pad(x, ((0, 0), (0, 0), (0, NPAD - N)))
    y_p = jnp.pad(y, ((0, 0), (0, 0), (0, NPAD - N)))

    xyzt = jnp.transpose(y_xyz, (0, 2, 1))
    txp = jnp.pad(xyzt, ((0, 0), (0, NPAD - N), (0, 13))).reshape(B * NPAD, 16)

    xqp, tkx, tv = _projections(
        x_p, y_p, txp, Wq, bq.reshape(1, QK).astype(f32), Wk,
        bk.reshape(1, QK).astype(f32), Wv, bv.reshape(1, C).astype(f32))
    xq = xqp.reshape(B, NPAD, QK)[:, :N].reshape(F, QK)

    gidx = (idx.astype(jnp.int32)
            + (jnp.arange(B, dtype=jnp.int32) * NPAD)[:, None, None]
            ).reshape(FS)
    gidxp = gidx.reshape(FP, 4).T.reshape(FS)

    gkp, gu = _sc_gather_kx(gidx, tkx)
    gv = _sc_gather_v(gidxp, tv)
    gv4 = gv.reshape(4, FP, C)

    eye4 = jnp.eye(4, dtype=f32)
    w1blk = jnp.kron(eye4, jnp.pad(geW1.T, ((0, QK - 4), (0, 0))))
    w2blk = jnp.kron(eye4, geW2.T)
    nw1blk = jnp.kron(eye4, neW1.T)
    w2cat = jnp.concatenate(
        [jnp.pad(neW2.T, ((QK * j, QK * (3 - j)), (0, 0))) for j in range(4)],
        axis=1)
    mblk = jnp.zeros((QK, QK), f32).at[0:3, 3].set(1.0)
    msum = jnp.kron(eye4, mblk)
    nb2cat = jnp.tile(neb2.reshape(1, C), (1, 4))

    gu_spec = pl.BlockSpec((T4, C), lambda i: (i, 0))
    stats_spec = pl.BlockSpec((8, C), lambda i: (0, 0))
    stats_shape = jax.ShapeDtypeStruct((8, C), f32)

    p_mat, su = _pcall(
        _c1_body,
        grid=(GRID,),
        in_specs=[gu_spec, _full((C, C))],
        out_specs=[pl.BlockSpec((C, C), lambda i: (0, 0)), stats_spec],
        out_shape=[jax.ShapeDtypeStruct((C, C), f32), stats_shape],
    )(gu, msum)

    eu = _fold(su[0])[:4] / MF
    p4 = jnp.einsum('aiaj->ij', p_mat.reshape(4, QK, 4, QK))[:4, :4] / MF
    cov = p4 - jnp.outer(eu, eu)
    mu1 = geW1 @ eu + geb1
    var1 = jnp.einsum('oi,ij,oj->o', geW1, cov, geW1)
    a1 = geg1 * lax.rsqrt(var1 + 1e-5)
    c1 = gebe1 - mu1 * a1

    emb, s2 = _pcall(
        _c2_body,
        grid=(GRID,),
        in_specs=[gu_spec, gu_spec, _full((C, C)),
                  pl.BlockSpec((TILE, QK), lambda i: (i, 0)),
                  _full((C, C)), _full((1, C)), _full((1, C)), _full((1, C)),
                  _full((C, C)), _full((1, C))],
        out_specs=[gu_spec, stats_spec],
        out_shape=[jax.ShapeDtypeStruct((FP, C), f32), stats_shape],
    )(gkp, gu, msum, xq, w1blk, _tile4(geb1), _tile4(a1), _tile4(c1), w2blk,
      _tile4(geb2))

    mu2 = _fold(s2[0]) / MF
    var2 = _fold(s2[1]) / MF - mu2 * mu2
    a2 = neg0 * lax.rsqrt(var2 + 1e-5)
    c2 = nebe0 - mu2 * a2

    s3 = _pcall(
        _c3_body,
        grid=(GRID,),
        in_specs=[gu_spec, _full((1, C)), _full((1, C)), _full((C, C)),
                  _full((1, C))],
        out_specs=stats_spec,
        out_shape=stats_shape,
    )(emb, _tile4(a2), _tile4(c2), nw1blk, _tile4(neb1))

    mu3 = _fold(s3[0]) / MF
    var3 = _fold(s3[1]) / MF - mu3 * mu3
    a3 = neg1 * lax.rsqrt(var3 + 1e-5)
    c3 = nebe1 - mu3 * a3

    out = _pcall(
        _c4_body,
        grid=(GRID,),
        in_specs=([gu_spec]
                  + [pl.BlockSpec((1, T4, C), lambda i, j=j: (j, i, 0))
                     for j in range(4)]
                  + [_full((1, C)), _full((1, C)), _full((C, C)),
                     _full((1, C)), _full((1, C)), _full((1, C)),
                     _full((C, 4 * C)), _full((1, 4 * C))]),
        out_specs=pl.BlockSpec((TILE, C), lambda i: (i, 0)),
        out_shape=jax.ShapeDtypeStruct((F, C), f32),
    )(emb, gv4, gv4, gv4, gv4, _tile4(a2), _tile4(c2), nw1blk, _tile4(neb1),
      _tile4(a3), _tile4(c3), w2cat, nb2cat)

    return out.reshape(B, N, C).transpose(0, 2, 1)
